# baseline (speedup 1.0000x reference)
import functools

import jax
import jax.numpy as jnp
from jax import lax
from jax.experimental import pallas as pl
from jax.experimental.pallas import tpu as pltpu
from jax.experimental.pallas import tpu_sc as plsc

N = 10000
E = 320000
HID = 128
BOX = 27.27
POS_MEAN = 13.635
POS_VAR = 61.97

NC = 2
NS = 16
NW = NC * NS
CH = 80
NB = 5
F32 = jnp.float32


def _softplus(x):
    return jnp.maximum(x, 0.0) + jnp.log1p(jnp.exp(-jnp.abs(x)))


def _make_sc_gather(n_rows, d, tc_tiling=True):
    per_w = n_rows // NW
    nch = per_w // CH
    assert per_w % CH == 0 and nch % NB == 0
    mesh = plsc.VectorSubcoreMesh(core_axis_name="c", subcore_axis_name="s")

    def body(table_hbm, idx_hbm, out_hbm, idx_v, *rest):
        bufs = rest[:NB]
        sems = rest[NB:]
        wid = lax.axis_index("s") * NC + lax.axis_index("c")
        base = pl.multiple_of(wid * per_w, 8)
        pltpu.sync_copy(idx_hbm.at[pl.ds(base, per_w)], idx_v)

        def step(i, carry):
            j0 = i * NB
            cps = []
            for b in range(NB):
                st = pl.multiple_of((j0 + b) * CH, 8)
                cps.append(pltpu.async_copy(
                    table_hbm.at[idx_v.at[pl.ds(st, CH)]], bufs[b], sems[b]))
            for b in range(NB):
                st = pl.multiple_of((j0 + b) * CH, 8)
                cps[b].wait()
                pltpu.sync_copy(bufs[b], out_hbm.at[pl.ds(base + st, CH)])
            return carry

        lax.fori_loop(0, nch // NB, step, 0)

    return pl.kernel(
        body,
        out_type=jax.ShapeDtypeStruct((n_rows, d), F32),
        mesh=mesh,
        scratch_types=(
            [pltpu.VMEM((per_w,), jnp.int32)]
            + [pltpu.VMEM((CH, d), F32) for _ in range(NB)]
            + [pltpu.SemaphoreType.DMA for _ in range(NB)]
        ),
        compiler_params=pltpu.CompilerParams(use_tc_tiling_on_sc=tc_tiling),
    )


HC = HID // 2
ZRS = 624
ZRL = 640


def _make_sc_scatter():
    per_t = E // NS
    nch = per_t // CH
    assert nch % NB == 0
    mesh = plsc.VectorSubcoreMesh(core_axis_name="c", subcore_axis_name="s")

    def body(m_hbm, dst_hbm, z_hbm, out_hbm, idx_v, *rest):
        bufs = rest[:NB]
        acc = rest[NB]
        sems = rest[NB + 1:]
        cid = lax.axis_index("c")
        sid = lax.axis_index("s")
        zbase = pl.multiple_of(sid * ZRS, 8)
        pltpu.sync_copy(z_hbm, acc.at[pl.ds(zbase, ZRL)])
        pltpu.sync_copy(dst_hbm.at[sid], idx_v)
        plsc.subcore_barrier()
        ebase = sid * per_t

        def step(i, carry):
            j0 = i * NB
            cps = []
            for b in range(NB):
                st = pl.multiple_of(ebase + (j0 + b) * CH, 8)
                cps.append(pltpu.async_copy(
                    m_hbm.at[cid, pl.ds(st, CH)], bufs[b], sems[b]))
            for b in range(NB):
                cps[b].wait()
                pltpu.sync_copy(bufs[b], acc.at[idx_v.at[j0 + b]], add=True)
            return carry

        lax.fori_loop(0, nch // NB, step, 0)
        plsc.subcore_barrier()
        pltpu.sync_copy(acc.at[pl.ds(zbase, ZRL)],
                        out_hbm.at[cid, pl.ds(zbase, ZRL)])

    return pl.kernel(
        body,
        out_type=jax.ShapeDtypeStruct((NC, N, HC), F32),
        mesh=mesh,
        scratch_types=(
            [pltpu.VMEM((nch, CH), jnp.int32)]
            + [pltpu.VMEM((CH, HC), F32) for _ in range(NB)]
            + [pltpu.VMEM_SHARED((N, HC), F32)]
            + [pltpu.SemaphoreType.DMA for _ in range(NB)]
        ),
        compiler_params=pltpu.CompilerParams(use_tc_tiling_on_sc=False),
    )


_BN = 1000
_BE = 4000

def _agg_specs():
    return [pl.BlockSpec((1, _BN, HC), lambda i: (0, i, 0)),
            pl.BlockSpec((1, _BN, HC), lambda i: (1, i, 0))]


def _full(spec_shape):
    return pl.BlockSpec(spec_shape, lambda i: tuple(0 for _ in spec_shape))


def _tc_prep(pos_vel, enc_W8, enc_b, mt0):
    def body(pv_ref, w_ref, b_ref, mt_ref, p16_ref, h_ref, p_ref):
        pv = pv_ref[...]
        pos = pv[:, 0:3] * jnp.sqrt(jnp.float32(POS_VAR)) + jnp.float32(POS_MEAN)
        z = jnp.zeros((pos.shape[0], 13), F32)
        p16_ref[...] = jnp.concatenate([pos, z], axis=1)
        pos8 = jnp.concatenate([pos, z[:, :5]], axis=1)
        h = _softplus(jnp.dot(pos8, w_ref[...], preferred_element_type=F32)
                      + b_ref[...])
        h_ref[...] = h
        p_ref[...] = jnp.dot(h, mt_ref[...], preferred_element_type=F32)

    grid = (N // _BN,)
    return pl.pallas_call(
        body,
        grid=grid,
        in_specs=[
            pl.BlockSpec((_BN, 6), lambda i: (i, 0)),
            _full((8, HID)),
            _full((1, HID)),
            _full((HID, HID)),
        ],
        out_specs=[
            pl.BlockSpec((_BN, 16), lambda i: (i, 0)),
            pl.BlockSpec((_BN, HID), lambda i: (i, 0)),
            pl.BlockSpec((_BN, HID), lambda i: (i, 0)),
        ],
        out_shape=[
            jax.ShapeDtypeStruct((N, 16), F32),
            jax.ShapeDtypeStruct((N, HID), F32),
            jax.ShapeDtypeStruct((N, HID), F32),
        ],
        compiler_params=pltpu.CompilerParams(
            dimension_semantics=("parallel",)),
    )(pos_vel, enc_W8, enc_b, mt0)


def _tc_edge_feat(pp):
    def body(ps_ref, pd_ref, ef_ref):
        d = ps_ref[...] - pd_ref[...]
        d = d - jnp.float32(BOX) * jnp.round(d / jnp.float32(BOX))
        ssum = jnp.sum(d * d, axis=1, keepdims=True) + jnp.float32(1e-12)
        dist = jnp.sqrt(ssum)
        col = lax.broadcasted_iota(jnp.int32, d.shape, 1)
        ef_ref[...] = jnp.where(col == 3, dist, d)

    grid = (E // _BE,)
    return pl.pallas_call(
        body,
        grid=grid,
        in_specs=[
            pl.BlockSpec((_BE, 16), lambda i: (i, 0)),
            pl.BlockSpec((_BE, 16), lambda i: (i + E // _BE, 0)),
        ],
        out_specs=pl.BlockSpec((_BE, 16), lambda i: (i, 0)),
        out_shape=jax.ShapeDtypeStruct((E, 16), F32),
        compiler_params=pltpu.CompilerParams(
            dimension_semantics=("parallel",)),
    )(pp, pp)


def _tc_edge_msg(g, ef, eW16, eb, wb, mb):
    def body(g_ref, ef_ref, ew_ref, eb_ref, wb_ref, mb_ref, m_ref):
        e = _softplus(jnp.dot(ef_ref[...], ew_ref[...],
                              preferred_element_type=F32) + eb_ref[...])
        m = _softplus(
            g_ref[...] + jnp.dot(e, wb_ref[...], preferred_element_type=F32)
            + mb_ref[...])
        m_ref[0] = m[:, :HC]
        m_ref[1] = m[:, HC:]

    grid = (E // _BE,)
    return pl.pallas_call(
        body,
        grid=grid,
        in_specs=[
            pl.BlockSpec((_BE, HID), lambda i: (i, 0)),
            pl.BlockSpec((_BE, 16), lambda i: (i, 0)),
            _full((16, HID)),
            _full((1, HID)),
            _full((HID, HID)),
            _full((1, HID)),
        ],
        out_specs=pl.BlockSpec((2, _BE, HC), lambda i: (0, i, 0)),
        out_shape=jax.ShapeDtypeStruct((2, E, HC), F32),
        compiler_params=pltpu.CompilerParams(
            dimension_semantics=("parallel",)),
    )(g, ef, eW16, eb, wb, mb)


def _ln_update(h, agg, ul_ref, ur_ref, ub_ref, g_ref, b_ref):
    u = _softplus(jnp.dot(h, ul_ref[...], preferred_element_type=F32)
                  + jnp.dot(agg, ur_ref[...], preferred_element_type=F32)
                  + ub_ref[...])
    hn = h + u
    mu = jnp.mean(hn, axis=1, keepdims=True)
    var = jnp.mean((hn - mu) * (hn - mu), axis=1, keepdims=True)
    return (hn - mu) / jnp.sqrt(var + jnp.float32(1e-5)) * g_ref[...] + b_ref[...]


def _tc_update(h, aggp, ul, ur, ub, lng, lnb, mt_next):
    def body(h_ref, a0_ref, a1_ref, ul_ref, ur_ref, ub_ref, g_ref, b_ref,
             mt_ref, h_out, p_out):
        agg = jnp.concatenate([a0_ref[0], a1_ref[0]], axis=1)
        hn = _ln_update(h_ref[...], agg, ul_ref, ur_ref, ub_ref,
                        g_ref, b_ref)
        h_out[...] = hn
        p_out[...] = jnp.dot(hn, mt_ref[...], preferred_element_type=F32)

    grid = (N // _BN,)
    return pl.pallas_call(
        body,
        grid=grid,
        in_specs=[
            pl.BlockSpec((_BN, HID), lambda i: (i, 0)),
            *_agg_specs(),
            _full((HID, HID)),
            _full((HID, HID)),
            _full((1, HID)),
            _full((1, HID)),
            _full((1, HID)),
            _full((HID, HID)),
        ],
        out_specs=[
            pl.BlockSpec((_BN, HID), lambda i: (i, 0)),
            pl.BlockSpec((_BN, HID), lambda i: (i, 0)),
        ],
        out_shape=[
            jax.ShapeDtypeStruct((N, HID), F32),
            jax.ShapeDtypeStruct((N, HID), F32),
        ],
        compiler_params=pltpu.CompilerParams(
            dimension_semantics=("parallel",)),
    )(h, aggp, aggp, ul, ur, ub, lng, lnb, mt_next)


def _tc_final(h, aggp, ul, ur, ub, lng, lnb, d1, db1, d2p, db2, f1p, fb1,
              f2, fb2, f3p, fb3p, pos_vel):
    def body(h_ref, a0_ref, a1_ref, ul_ref, ur_ref, ub_ref, g_ref, b_ref,
             d1_ref, db1_ref, d2_ref, db2_ref, f1_ref, fb1_ref, f2_ref,
             fb2_ref, f3_ref, fb3_ref, pv_ref, out_ref):
        agg = jnp.concatenate([a0_ref[0], a1_ref[0]], axis=1)
        hn = _ln_update(h_ref[...], agg, ul_ref, ur_ref, ub_ref,
                        g_ref, b_ref)
        fmid = _softplus(jnp.dot(hn, d1_ref[...], preferred_element_type=F32)
                         + db1_ref[...])
        force = jnp.dot(fmid, d2_ref[...], preferred_element_type=F32) \
            + db2_ref[...]
        pv = pv_ref[...]
        x16 = jnp.concatenate(
            [pv, force[:, 0:3], jnp.zeros((pv.shape[0], 7), F32)], axis=1)
        x = _softplus(jnp.dot(x16, f1_ref[...], preferred_element_type=F32)
                      + fb1_ref[...])
        x = _softplus(jnp.dot(x, f2_ref[...], preferred_element_type=F32)
                      + fb2_ref[...])
        out = jnp.dot(x, f3_ref[...], preferred_element_type=F32) + fb3_ref[...]
        out_ref[...] = out[:, 0:6]

    grid = (N // _BN,)
    return pl.pallas_call(
        body,
        grid=grid,
        in_specs=[
            pl.BlockSpec((_BN, HID), lambda i: (i, 0)),
            *_agg_specs(),
            _full((HID, HID)),
            _full((HID, HID)),
            _full((1, HID)),
            _full((1, HID)),
            _full((1, HID)),
            _full((HID, HID)),
            _full((1, HID)),
            _full((HID, 8)),
            _full((1, 8)),
            _full((16, HID)),
            _full((1, HID)),
            _full((HID, HID)),
            _full((1, HID)),
            _full((HID, 8)),
            _full((1, 8)),
            pl.BlockSpec((_BN, 6), lambda i: (i, 0)),
        ],
        out_specs=pl.BlockSpec((_BN, 6), lambda i: (i, 0)),
        out_shape=jax.ShapeDtypeStruct((N, 6), F32),
        compiler_params=pltpu.CompilerParams(
            dimension_semantics=("parallel",)),
    )(h, aggp, aggp, ul, ur, ub, lng, lnb, d1, db1, d2p, db2, f1p, fb1,
      f2, fb2, f3p, fb3p, pos_vel)


def kernel(pos_vel, t, edge_index, enc_W, enc_b, edge_W, edge_b, msg_W, msg_b,
           upd_W, upd_b, ln_g, ln_b, dec_W1, dec_b1, dec_W2, dec_b2, fc_W1,
           fc_b1, fc_W2, fc_b2, fc_W3, fc_b3):
    src = edge_index[0]
    dst = edge_index[1]

    enc_W8 = jnp.pad(enc_W, ((0, 5), (0, 0)))
    eW16 = jnp.pad(edge_W, ((0, 12), (0, 0)))
    row = lambda v: v.reshape(1, -1)
    mt = [msg_W[l][:HID] for l in range(4)]
    mbot = [msg_W[l][HID:] for l in range(4)]
    ul = [upd_W[l][:HID] for l in range(4)]
    ur = [upd_W[l][HID:] for l in range(4)]
    d2p = jnp.pad(dec_W2, ((0, 0), (0, 5)))
    db2 = jnp.pad(dec_b2, (0, 5))
    f1p = jnp.pad(fc_W1, ((0, 7), (0, 0)))
    f3p = jnp.pad(fc_W3, ((0, 0), (0, 2)))
    fb3p = jnp.pad(fc_b3, (0, 2))

    p16, h, p = _tc_prep(pos_vel, enc_W8, row(enc_b), mt[0])

    gather16 = _make_sc_gather(2 * E, 16, tc_tiling=False)
    pp = gather16(p16, jnp.concatenate([src, dst]))
    ef = _tc_edge_feat(pp)

    gather128 = _make_sc_gather(E, HID)
    scatter = _make_sc_scatter()
    dst3 = dst.reshape(NS, (E // NS) // CH, CH)
    zeros = jnp.zeros((ZRL, HC), F32)

    out = None
    for l in range(4):
        g = gather128(p, src)
        m = _tc_edge_msg(g, ef, eW16, row(edge_b), mbot[l], row(msg_b[l]))
        aggp = scatter(m, dst3, zeros)
        if l < 3:
            h, p = _tc_update(h, aggp, ul[l], ur[l], row(upd_b[l]),
                              row(ln_g[l]), row(ln_b[l]), mt[l + 1])
        else:
            out = _tc_final(h, aggp, ul[l], ur[l], row(upd_b[l]),
                            row(ln_g[l]), row(ln_b[l]), dec_W1, row(dec_b1),
                            d2p, row(db2), f1p, row(fc_b1), fc_W2, row(fc_b2),
                            f3p, row(fb3p), pos_vel)
    return out


# speedup vs baseline: 1.4375x; 1.4375x over previous
import functools

import jax
import jax.numpy as jnp
from jax import lax
from jax.experimental import pallas as pl
from jax.experimental.pallas import tpu as pltpu
from jax.experimental.pallas import tpu_sc as plsc

N = 10000
E = 320000
HID = 128
BOX = 27.27
POS_MEAN = 13.635
POS_VAR = 61.97

NC = 2
NS = 16
NW = NC * NS
CH = 80
NB = 5
F32 = jnp.float32


def _softplus(x):
    return jnp.maximum(x, 0.0) + jnp.log1p(jnp.exp(-jnp.abs(x)))


def _make_sc_gather(n_rows, d, tc_tiling=True):
    per_w = n_rows // NW
    nch = per_w // CH
    assert per_w % CH == 0 and nch % NB == 0
    mesh = plsc.VectorSubcoreMesh(core_axis_name="c", subcore_axis_name="s")

    def body(table_hbm, idx_hbm, out_hbm, idx_v, *rest):
        bufs = rest[:NB]
        sems = rest[NB:]
        wid = lax.axis_index("s") * NC + lax.axis_index("c")
        base = pl.multiple_of(wid * per_w, 8)
        pltpu.sync_copy(idx_hbm.at[pl.ds(base, per_w)], idx_v)

        def step(i, carry):
            j0 = i * NB
            cps = []
            for b in range(NB):
                st = pl.multiple_of((j0 + b) * CH, 8)
                cps.append(pltpu.async_copy(
                    table_hbm.at[idx_v.at[pl.ds(st, CH)]], bufs[b], sems[b]))
            for b in range(NB):
                st = pl.multiple_of((j0 + b) * CH, 8)
                cps[b].wait()
                pltpu.sync_copy(bufs[b], out_hbm.at[pl.ds(base + st, CH)])
            return carry

        lax.fori_loop(0, nch // NB, step, 0)

    return pl.kernel(
        body,
        out_type=jax.ShapeDtypeStruct((n_rows, d), F32),
        mesh=mesh,
        scratch_types=(
            [pltpu.VMEM((per_w,), jnp.int32)]
            + [pltpu.VMEM((CH, d), F32) for _ in range(NB)]
            + [pltpu.SemaphoreType.DMA for _ in range(NB)]
        ),
        compiler_params=pltpu.CompilerParams(use_tc_tiling_on_sc=tc_tiling),
    )


HC = HID // 2
ZRS = 624
ZRL = 640


def _make_sc_scatter():
    per_t = E // NS
    nch = per_t // CH
    assert nch % NB == 0
    mesh = plsc.VectorSubcoreMesh(core_axis_name="c", subcore_axis_name="s")

    def body(m_hbm, dst_hbm, z_hbm, out_hbm, idx_v, *rest):
        bufs = rest[:NB]
        acc = rest[NB]
        sems = rest[NB + 1:]
        cid = lax.axis_index("c")
        sid = lax.axis_index("s")
        zbase = pl.multiple_of(sid * ZRS, 8)
        pltpu.sync_copy(z_hbm, acc.at[pl.ds(zbase, ZRL)])
        pltpu.sync_copy(dst_hbm.at[sid], idx_v)
        plsc.subcore_barrier()
        ebase = sid * per_t

        cb = pl.multiple_of(cid * HC, 8)

        def step(i, carry):
            j0 = i * NB
            cps = []
            for b in range(NB):
                st = pl.multiple_of(ebase + (j0 + b) * CH, 8)
                cps.append(pltpu.async_copy(
                    m_hbm.at[pl.ds(st, CH), pl.ds(cb, HC)], bufs[b], sems[b]))
            for b in range(NB):
                cps[b].wait()
                pltpu.sync_copy(bufs[b], acc.at[idx_v.at[j0 + b]], add=True)
            return carry

        lax.fori_loop(0, nch // NB, step, 0)
        plsc.subcore_barrier()
        pltpu.sync_copy(acc.at[pl.ds(zbase, ZRL)],
                        out_hbm.at[cid, pl.ds(zbase, ZRL)])

    return pl.kernel(
        body,
        out_type=jax.ShapeDtypeStruct((NC, N, HC), F32),
        mesh=mesh,
        scratch_types=(
            [pltpu.VMEM((nch, CH), jnp.int32)]
            + [pltpu.VMEM((CH, HC), F32) for _ in range(NB)]
            + [pltpu.VMEM_SHARED((N, HC), F32)]
            + [pltpu.SemaphoreType.DMA for _ in range(NB)]
        ),
        compiler_params=pltpu.CompilerParams(use_tc_tiling_on_sc=False),
    )


_BN = 1000
_BE = 4000

def _agg_specs():
    return [pl.BlockSpec((1, _BN, HC), lambda i: (0, i, 0)),
            pl.BlockSpec((1, _BN, HC), lambda i: (1, i, 0))]


def _full(spec_shape):
    return pl.BlockSpec(spec_shape, lambda i: tuple(0 for _ in spec_shape))


def _tc_prep(pos_vel, enc_W8, enc_b, mt0):
    def body(pv_ref, w_ref, b_ref, mt_ref, p16_ref, h_ref, p_ref):
        pv = pv_ref[...]
        pos = pv[:, 0:3] * jnp.sqrt(jnp.float32(POS_VAR)) + jnp.float32(POS_MEAN)
        z = jnp.zeros((pos.shape[0], 13), F32)
        p16_ref[...] = jnp.concatenate([pos, z], axis=1)
        pos8 = jnp.concatenate([pos, z[:, :5]], axis=1)
        h = _softplus(jnp.dot(pos8, w_ref[...], preferred_element_type=F32)
                      + b_ref[...])
        h_ref[...] = h
        p_ref[...] = jnp.dot(h, mt_ref[...], preferred_element_type=F32)

    grid = (N // _BN,)
    return pl.pallas_call(
        body,
        grid=grid,
        in_specs=[
            pl.BlockSpec((_BN, 6), lambda i: (i, 0)),
            _full((8, HID)),
            _full((1, HID)),
            _full((HID, HID)),
        ],
        out_specs=[
            pl.BlockSpec((_BN, 16), lambda i: (i, 0)),
            pl.BlockSpec((_BN, HID), lambda i: (i, 0)),
            pl.BlockSpec((_BN, HID), lambda i: (i, 0)),
        ],
        out_shape=[
            jax.ShapeDtypeStruct((N, 16), F32),
            jax.ShapeDtypeStruct((N, HID), F32),
            jax.ShapeDtypeStruct((N, HID), F32),
        ],
        compiler_params=pltpu.CompilerParams(
            dimension_semantics=("parallel",)),
    )(pos_vel, enc_W8, enc_b, mt0)


def _tc_edge_feat(pp):
    def body(ps_ref, pd_ref, ef_ref):
        d = ps_ref[...] - pd_ref[...]
        d = d - jnp.float32(BOX) * jnp.round(d / jnp.float32(BOX))
        ssum = jnp.sum(d * d, axis=1, keepdims=True) + jnp.float32(1e-12)
        dist = jnp.sqrt(ssum)
        col = lax.broadcasted_iota(jnp.int32, d.shape, 1)
        ef_ref[...] = jnp.where(col == 3, dist, d)

    grid = (E // _BE,)
    return pl.pallas_call(
        body,
        grid=grid,
        in_specs=[
            pl.BlockSpec((_BE, 16), lambda i: (i, 0)),
            pl.BlockSpec((_BE, 16), lambda i: (i + E // _BE, 0)),
        ],
        out_specs=pl.BlockSpec((_BE, 16), lambda i: (i, 0)),
        out_shape=jax.ShapeDtypeStruct((E, 16), F32),
        compiler_params=pltpu.CompilerParams(
            dimension_semantics=("parallel",)),
    )(pp, pp)


def _tc_edge_msg(g, ef, eW16, eb, wb, mb):
    def body(g_ref, ef_ref, ew_ref, eb_ref, wb_ref, mb_ref, m_ref):
        e = _softplus(jnp.dot(ef_ref[...], ew_ref[...],
                              preferred_element_type=F32) + eb_ref[...])
        m_ref[...] = _softplus(
            g_ref[...] + jnp.dot(e, wb_ref[...], preferred_element_type=F32)
            + mb_ref[...])

    grid = (E // _BE,)
    return pl.pallas_call(
        body,
        grid=grid,
        in_specs=[
            pl.BlockSpec((_BE, HID), lambda i: (i, 0)),
            pl.BlockSpec((_BE, 16), lambda i: (i, 0)),
            _full((16, HID)),
            _full((1, HID)),
            _full((HID, HID)),
            _full((1, HID)),
        ],
        out_specs=pl.BlockSpec((_BE, HID), lambda i: (i, 0)),
        out_shape=jax.ShapeDtypeStruct((E, HID), F32),
        compiler_params=pltpu.CompilerParams(
            dimension_semantics=("parallel",)),
    )(g, ef, eW16, eb, wb, mb)


def _ln_update(h, agg, ul_ref, ur_ref, ub_ref, g_ref, b_ref):
    u = _softplus(jnp.dot(h, ul_ref[...], preferred_element_type=F32)
                  + jnp.dot(agg, ur_ref[...], preferred_element_type=F32)
                  + ub_ref[...])
    hn = h + u
    mu = jnp.mean(hn, axis=1, keepdims=True)
    var = jnp.mean((hn - mu) * (hn - mu), axis=1, keepdims=True)
    return (hn - mu) / jnp.sqrt(var + jnp.float32(1e-5)) * g_ref[...] + b_ref[...]


def _tc_update(h, aggp, ul, ur, ub, lng, lnb, mt_next):
    def body(h_ref, a0_ref, a1_ref, ul_ref, ur_ref, ub_ref, g_ref, b_ref,
             mt_ref, h_out, p_out):
        agg = jnp.concatenate([a0_ref[0], a1_ref[0]], axis=1)
        hn = _ln_update(h_ref[...], agg, ul_ref, ur_ref, ub_ref,
                        g_ref, b_ref)
        h_out[...] = hn
        p_out[...] = jnp.dot(hn, mt_ref[...], preferred_element_type=F32)

    grid = (N // _BN,)
    return pl.pallas_call(
        body,
        grid=grid,
        in_specs=[
            pl.BlockSpec((_BN, HID), lambda i: (i, 0)),
            *_agg_specs(),
            _full((HID, HID)),
            _full((HID, HID)),
            _full((1, HID)),
            _full((1, HID)),
            _full((1, HID)),
            _full((HID, HID)),
        ],
        out_specs=[
            pl.BlockSpec((_BN, HID), lambda i: (i, 0)),
            pl.BlockSpec((_BN, HID), lambda i: (i, 0)),
        ],
        out_shape=[
            jax.ShapeDtypeStruct((N, HID), F32),
            jax.ShapeDtypeStruct((N, HID), F32),
        ],
        compiler_params=pltpu.CompilerParams(
            dimension_semantics=("parallel",)),
    )(h, aggp, aggp, ul, ur, ub, lng, lnb, mt_next)


def _tc_final(h, aggp, ul, ur, ub, lng, lnb, d1, db1, d2p, db2, f1p, fb1,
              f2, fb2, f3p, fb3p, pos_vel):
    def body(h_ref, a0_ref, a1_ref, ul_ref, ur_ref, ub_ref, g_ref, b_ref,
             d1_ref, db1_ref, d2_ref, db2_ref, f1_ref, fb1_ref, f2_ref,
             fb2_ref, f3_ref, fb3_ref, pv_ref, out_ref):
        agg = jnp.concatenate([a0_ref[0], a1_ref[0]], axis=1)
        hn = _ln_update(h_ref[...], agg, ul_ref, ur_ref, ub_ref,
                        g_ref, b_ref)
        fmid = _softplus(jnp.dot(hn, d1_ref[...], preferred_element_type=F32)
                         + db1_ref[...])
        force = jnp.dot(fmid, d2_ref[...], preferred_element_type=F32) \
            + db2_ref[...]
        pv = pv_ref[...]
        x16 = jnp.concatenate(
            [pv, force[:, 0:3], jnp.zeros((pv.shape[0], 7), F32)], axis=1)
        x = _softplus(jnp.dot(x16, f1_ref[...], preferred_element_type=F32)
                      + fb1_ref[...])
        x = _softplus(jnp.dot(x, f2_ref[...], preferred_element_type=F32)
                      + fb2_ref[...])
        out = jnp.dot(x, f3_ref[...], preferred_element_type=F32) + fb3_ref[...]
        out_ref[...] = out[:, 0:6]

    grid = (N // _BN,)
    return pl.pallas_call(
        body,
        grid=grid,
        in_specs=[
            pl.BlockSpec((_BN, HID), lambda i: (i, 0)),
            *_agg_specs(),
            _full((HID, HID)),
            _full((HID, HID)),
            _full((1, HID)),
            _full((1, HID)),
            _full((1, HID)),
            _full((HID, HID)),
            _full((1, HID)),
            _full((HID, 8)),
            _full((1, 8)),
            _full((16, HID)),
            _full((1, HID)),
            _full((HID, HID)),
            _full((1, HID)),
            _full((HID, 8)),
            _full((1, 8)),
            pl.BlockSpec((_BN, 6), lambda i: (i, 0)),
        ],
        out_specs=pl.BlockSpec((_BN, 6), lambda i: (i, 0)),
        out_shape=jax.ShapeDtypeStruct((N, 6), F32),
        compiler_params=pltpu.CompilerParams(
            dimension_semantics=("parallel",)),
    )(h, aggp, aggp, ul, ur, ub, lng, lnb, d1, db1, d2p, db2, f1p, fb1,
      f2, fb2, f3p, fb3p, pos_vel)


def kernel(pos_vel, t, edge_index, enc_W, enc_b, edge_W, edge_b, msg_W, msg_b,
           upd_W, upd_b, ln_g, ln_b, dec_W1, dec_b1, dec_W2, dec_b2, fc_W1,
           fc_b1, fc_W2, fc_b2, fc_W3, fc_b3):
    src = edge_index[0]
    dst = edge_index[1]

    enc_W8 = jnp.pad(enc_W, ((0, 5), (0, 0)))
    eW16 = jnp.pad(edge_W, ((0, 12), (0, 0)))
    row = lambda v: v.reshape(1, -1)
    mt = [msg_W[l][:HID] for l in range(4)]
    mbot = [msg_W[l][HID:] for l in range(4)]
    ul = [upd_W[l][:HID] for l in range(4)]
    ur = [upd_W[l][HID:] for l in range(4)]
    d2p = jnp.pad(dec_W2, ((0, 0), (0, 5)))
    db2 = jnp.pad(dec_b2, (0, 5))
    f1p = jnp.pad(fc_W1, ((0, 7), (0, 0)))
    f3p = jnp.pad(fc_W3, ((0, 0), (0, 2)))
    fb3p = jnp.pad(fc_b3, (0, 2))

    p16, h, p = _tc_prep(pos_vel, enc_W8, row(enc_b), mt[0])

    gather16 = _make_sc_gather(2 * E, 16, tc_tiling=False)
    pp = gather16(p16, jnp.concatenate([src, dst]))
    ef = _tc_edge_feat(pp)

    gather128 = _make_sc_gather(E, HID)
    scatter = _make_sc_scatter()
    dst3 = dst.reshape(NS, (E // NS) // CH, CH)
    zeros = jnp.zeros((ZRL, HC), F32)

    out = None
    for l in range(4):
        g = gather128(p, src)
        m = _tc_edge_msg(g, ef, eW16, row(edge_b), mbot[l], row(msg_b[l]))
        aggp = scatter(m, dst3, zeros)
        if l < 3:
            h, p = _tc_update(h, aggp, ul[l], ur[l], row(upd_b[l]),
                              row(ln_g[l]), row(ln_b[l]), mt[l + 1])
        else:
            out = _tc_final(h, aggp, ul[l], ur[l], row(upd_b[l]),
                            row(ln_g[l]), row(ln_b[l]), dec_W1, row(dec_b1),
                            d2p, row(db2), f1p, row(fc_b1), fc_W2, row(fc_b2),
                            f3p, row(fb3p), pos_vel)
    return out


# speedup vs baseline: 1.5940x; 1.1088x over previous
import functools

import jax
import jax.numpy as jnp
from jax import lax
from jax.experimental import pallas as pl
from jax.experimental.pallas import tpu as pltpu
from jax.experimental.pallas import tpu_sc as plsc

N = 10000
E = 320000
HID = 128
BOX = 27.27
POS_MEAN = 13.635
POS_VAR = 61.97

NC = 2
NS = 16
NW = NC * NS
CH = 80
NB = 5
F32 = jnp.float32


def _softplus(x):
    return jnp.maximum(x, 0.0) + jnp.log1p(jnp.exp(-jnp.abs(x)))


def _make_sc_gather(n_rows, d, ch=CH, tc_tiling=True):
    per_w = n_rows // NW
    nch = per_w // ch
    assert per_w % ch == 0 and nch % NB == 0
    mesh = plsc.VectorSubcoreMesh(core_axis_name="c", subcore_axis_name="s")

    def body(table_hbm, idx_hbm, out_hbm, idx_v, *rest):
        bufs = rest[:NB]
        sems = rest[NB:]
        wid = lax.axis_index("s") * NC + lax.axis_index("c")
        base = pl.multiple_of(wid * per_w, 8)
        pltpu.sync_copy(idx_hbm.at[pl.ds(base, per_w)], idx_v)

        def step(i, carry):
            j0 = i * NB
            cps = []
            for b in range(NB):
                st = pl.multiple_of((j0 + b) * ch, 8)
                cps.append(pltpu.async_copy(
                    table_hbm.at[idx_v.at[pl.ds(st, ch)]], bufs[b], sems[b]))
            for b in range(NB):
                st = pl.multiple_of((j0 + b) * ch, 8)
                cps[b].wait()
                pltpu.sync_copy(bufs[b], out_hbm.at[pl.ds(base + st, ch)])
            return carry

        lax.fori_loop(0, nch // NB, step, 0)

    return pl.kernel(
        body,
        out_type=jax.ShapeDtypeStruct((n_rows, d), F32),
        mesh=mesh,
        scratch_types=(
            [pltpu.VMEM((per_w,), jnp.int32)]
            + [pltpu.VMEM((ch, d), F32) for _ in range(NB)]
            + [pltpu.SemaphoreType.DMA for _ in range(NB)]
        ),
        compiler_params=pltpu.CompilerParams(use_tc_tiling_on_sc=tc_tiling),
    )


HC = HID // 2
ZRS = 624
ZRL = 640


def _make_sc_scatter(n_edges):
    per_t = n_edges // NS
    nch = per_t // CH
    assert nch % NB == 0
    mesh = plsc.VectorSubcoreMesh(core_axis_name="c", subcore_axis_name="s")

    def body(m_hbm, dst_hbm, z_hbm, out_hbm, idx_v, *rest):
        bufs = rest[:NB]
        acc = rest[NB]
        sems = rest[NB + 1:]
        cid = lax.axis_index("c")
        sid = lax.axis_index("s")
        zbase = pl.multiple_of(sid * ZRS, 8)
        pltpu.sync_copy(z_hbm, acc.at[pl.ds(zbase, ZRL)])
        pltpu.sync_copy(dst_hbm.at[sid], idx_v)
        plsc.subcore_barrier()
        ebase = sid * per_t

        cb = pl.multiple_of(cid * HC, 8)

        def step(i, carry):
            j0 = i * NB
            cps = []
            for b in range(NB):
                st = pl.multiple_of(ebase + (j0 + b) * CH, 8)
                cps.append(pltpu.async_copy(
                    m_hbm.at[pl.ds(st, CH), pl.ds(cb, HC)], bufs[b], sems[b]))
            for b in range(NB):
                cps[b].wait()
                pltpu.sync_copy(bufs[b], acc.at[idx_v.at[j0 + b]], add=True)
            return carry

        lax.fori_loop(0, nch // NB, step, 0)
        plsc.subcore_barrier()
        pltpu.sync_copy(acc.at[pl.ds(zbase, ZRL)],
                        out_hbm.at[cid, pl.ds(zbase, ZRL)])

    return pl.kernel(
        body,
        out_type=jax.ShapeDtypeStruct((NC, N, HC), F32),
        mesh=mesh,
        scratch_types=(
            [pltpu.VMEM((nch, CH), jnp.int32)]
            + [pltpu.VMEM((CH, HC), F32) for _ in range(NB)]
            + [pltpu.VMEM_SHARED((N, HC), F32)]
            + [pltpu.SemaphoreType.DMA for _ in range(NB)]
        ),
        compiler_params=pltpu.CompilerParams(use_tc_tiling_on_sc=False),
    )


_BN = 1000
_BE = 4000

def _agg_specs():
    return [pl.BlockSpec((1, _BN, HC), lambda i: (0, i, 0)),
            pl.BlockSpec((1, _BN, HC), lambda i: (1, i, 0))]


def _full(spec_shape):
    return pl.BlockSpec(spec_shape, lambda i: tuple(0 for _ in spec_shape))


def _tc_prep(pos_vel, enc_W8, enc_b, mt0):
    def body(pv_ref, w_ref, b_ref, mt_ref, p16_ref, h_ref, p_ref):
        pv = pv_ref[...]
        pos = pv[:, 0:3] * jnp.sqrt(jnp.float32(POS_VAR)) + jnp.float32(POS_MEAN)
        z = jnp.zeros((pos.shape[0], 13), F32)
        p16_ref[...] = jnp.concatenate([pos, z], axis=1)
        pos8 = jnp.concatenate([pos, z[:, :5]], axis=1)
        h = _softplus(jnp.dot(pos8, w_ref[...], preferred_element_type=F32)
                      + b_ref[...])
        h_ref[...] = h
        p_ref[...] = jnp.dot(h, mt_ref[...], preferred_element_type=F32)

    grid = (N // _BN,)
    return pl.pallas_call(
        body,
        grid=grid,
        in_specs=[
            pl.BlockSpec((_BN, 6), lambda i: (i, 0)),
            _full((8, HID)),
            _full((1, HID)),
            _full((HID, HID)),
        ],
        out_specs=[
            pl.BlockSpec((_BN, 16), lambda i: (i, 0)),
            pl.BlockSpec((_BN, HID), lambda i: (i, 0)),
            pl.BlockSpec((_BN, HID), lambda i: (i, 0)),
        ],
        out_shape=[
            jax.ShapeDtypeStruct((N, 16), F32),
            jax.ShapeDtypeStruct((N, HID), F32),
            jax.ShapeDtypeStruct((N, HID), F32),
        ],
        compiler_params=pltpu.CompilerParams(
            dimension_semantics=("parallel",)),
    )(pos_vel, enc_W8, enc_b, mt0)


def _tc_edge_feat(pp):
    def body(ps_ref, pd_ref, ef_ref):
        d = ps_ref[...] - pd_ref[...]
        d = d - jnp.float32(BOX) * jnp.round(d / jnp.float32(BOX))
        ssum = jnp.sum(d * d, axis=1, keepdims=True) + jnp.float32(1e-12)
        dist = jnp.sqrt(ssum)
        col = lax.broadcasted_iota(jnp.int32, d.shape, 1)
        ef_ref[...] = jnp.where(col == 3, dist, d)

    grid = (E // _BE,)
    return pl.pallas_call(
        body,
        grid=grid,
        in_specs=[
            pl.BlockSpec((_BE, 16), lambda i: (i, 0)),
            pl.BlockSpec((_BE, 16), lambda i: (i + E // _BE, 0)),
        ],
        out_specs=pl.BlockSpec((_BE, 16), lambda i: (i, 0)),
        out_shape=jax.ShapeDtypeStruct((E, 16), F32),
        compiler_params=pltpu.CompilerParams(
            dimension_semantics=("parallel",)),
    )(pp, pp)


def _tc_edge_msg(g, ef, koff, eW16, eb, wb, mb):
    n_edges = g.shape[0]

    def body(g_ref, ef_ref, ew_ref, eb_ref, wb_ref, mb_ref, m_ref):
        e = _softplus(jnp.dot(ef_ref[...], ew_ref[...],
                              preferred_element_type=F32) + eb_ref[...])
        m_ref[...] = _softplus(
            g_ref[...] + jnp.dot(e, wb_ref[...], preferred_element_type=F32)
            + mb_ref[...])

    grid = (n_edges // _BE,)
    return pl.pallas_call(
        body,
        grid=grid,
        in_specs=[
            pl.BlockSpec((_BE, HID), lambda i: (i, 0)),
            pl.BlockSpec((_BE, 16), lambda i: (i + koff, 0)),
            _full((16, HID)),
            _full((1, HID)),
            _full((HID, HID)),
            _full((1, HID)),
        ],
        out_specs=pl.BlockSpec((_BE, HID), lambda i: (i, 0)),
        out_shape=jax.ShapeDtypeStruct((n_edges, HID), F32),
        compiler_params=pltpu.CompilerParams(
            dimension_semantics=("parallel",)),
    )(g, ef, eW16, eb, wb, mb)


def _ln_update(h, agg, ul_ref, ur_ref, ub_ref, g_ref, b_ref):
    u = _softplus(jnp.dot(h, ul_ref[...], preferred_element_type=F32)
                  + jnp.dot(agg, ur_ref[...], preferred_element_type=F32)
                  + ub_ref[...])
    hn = h + u
    mu = jnp.mean(hn, axis=1, keepdims=True)
    var = jnp.mean((hn - mu) * (hn - mu), axis=1, keepdims=True)
    return (hn - mu) / jnp.sqrt(var + jnp.float32(1e-5)) * g_ref[...] + b_ref[...]


def _agg_cat(a0l_ref, a0r_ref, a1l_ref, a1r_ref):
    return jnp.concatenate([a0l_ref[0] + a1l_ref[0],
                            a0r_ref[0] + a1r_ref[0]], axis=1)


def _tc_update(h, agg0, agg1, ul, ur, ub, lng, lnb, mt_next):
    def body(h_ref, a0l_ref, a0r_ref, a1l_ref, a1r_ref, ul_ref, ur_ref,
             ub_ref, g_ref, b_ref, mt_ref, h_out, p_out):
        agg = _agg_cat(a0l_ref, a0r_ref, a1l_ref, a1r_ref)
        hn = _ln_update(h_ref[...], agg, ul_ref, ur_ref, ub_ref,
                        g_ref, b_ref)
        h_out[...] = hn
        p_out[...] = jnp.dot(hn, mt_ref[...], preferred_element_type=F32)

    grid = (N // _BN,)
    return pl.pallas_call(
        body,
        grid=grid,
        in_specs=[
            pl.BlockSpec((_BN, HID), lambda i: (i, 0)),
            *_agg_specs(),
            *_agg_specs(),
            _full((HID, HID)),
            _full((HID, HID)),
            _full((1, HID)),
            _full((1, HID)),
            _full((1, HID)),
            _full((HID, HID)),
        ],
        out_specs=[
            pl.BlockSpec((_BN, HID), lambda i: (i, 0)),
            pl.BlockSpec((_BN, HID), lambda i: (i, 0)),
        ],
        out_shape=[
            jax.ShapeDtypeStruct((N, HID), F32),
            jax.ShapeDtypeStruct((N, HID), F32),
        ],
        compiler_params=pltpu.CompilerParams(
            dimension_semantics=("parallel",)),
    )(h, agg0, agg0, agg1, agg1, ul, ur, ub, lng, lnb, mt_next)


def _tc_final(h, agg0, agg1, ul, ur, ub, lng, lnb, d1, db1, d2p, db2, f1p,
              fb1, f2, fb2, f3p, fb3p, pos_vel):
    def body(h_ref, a0l_ref, a0r_ref, a1l_ref, a1r_ref, ul_ref, ur_ref,
             ub_ref, g_ref, b_ref, d1_ref, db1_ref, d2_ref, db2_ref, f1_ref,
             fb1_ref, f2_ref, fb2_ref, f3_ref, fb3_ref, pv_ref, out_ref):
        agg = _agg_cat(a0l_ref, a0r_ref, a1l_ref, a1r_ref)
        hn = _ln_update(h_ref[...], agg, ul_ref, ur_ref, ub_ref,
                        g_ref, b_ref)
        fmid = _softplus(jnp.dot(hn, d1_ref[...], preferred_element_type=F32)
                         + db1_ref[...])
        force = jnp.dot(fmid, d2_ref[...], preferred_element_type=F32) \
            + db2_ref[...]
        pv = pv_ref[...]
        x16 = jnp.concatenate(
            [pv, force[:, 0:3], jnp.zeros((pv.shape[0], 7), F32)], axis=1)
        x = _softplus(jnp.dot(x16, f1_ref[...], preferred_element_type=F32)
                      + fb1_ref[...])
        x = _softplus(jnp.dot(x, f2_ref[...], preferred_element_type=F32)
                      + fb2_ref[...])
        out = jnp.dot(x, f3_ref[...], preferred_element_type=F32) + fb3_ref[...]
        out_ref[...] = out[:, 0:6]

    grid = (N // _BN,)
    return pl.pallas_call(
        body,
        grid=grid,
        in_specs=[
            pl.BlockSpec((_BN, HID), lambda i: (i, 0)),
            *_agg_specs(),
            *_agg_specs(),
            _full((HID, HID)),
            _full((HID, HID)),
            _full((1, HID)),
            _full((1, HID)),
            _full((1, HID)),
            _full((HID, HID)),
            _full((1, HID)),
            _full((HID, 8)),
            _full((1, 8)),
            _full((16, HID)),
            _full((1, HID)),
            _full((HID, HID)),
            _full((1, HID)),
            _full((HID, 8)),
            _full((1, 8)),
            pl.BlockSpec((_BN, 6), lambda i: (i, 0)),
        ],
        out_specs=pl.BlockSpec((_BN, 6), lambda i: (i, 0)),
        out_shape=jax.ShapeDtypeStruct((N, 6), F32),
        compiler_params=pltpu.CompilerParams(
            dimension_semantics=("parallel",)),
    )(h, agg0, agg0, agg1, agg1, ul, ur, ub, lng, lnb, d1, db1, d2p, db2,
      f1p, fb1, f2, fb2, f3p, fb3p, pos_vel)


def kernel(pos_vel, t, edge_index, enc_W, enc_b, edge_W, edge_b, msg_W, msg_b,
           upd_W, upd_b, ln_g, ln_b, dec_W1, dec_b1, dec_W2, dec_b2, fc_W1,
           fc_b1, fc_W2, fc_b2, fc_W3, fc_b3):
    src = edge_index[0]
    dst = edge_index[1]

    enc_W8 = jnp.pad(enc_W, ((0, 5), (0, 0)))
    eW16 = jnp.pad(edge_W, ((0, 12), (0, 0)))
    row = lambda v: v.reshape(1, -1)
    mt = [msg_W[l][:HID] for l in range(4)]
    mbot = [msg_W[l][HID:] for l in range(4)]
    ul = [upd_W[l][:HID] for l in range(4)]
    ur = [upd_W[l][HID:] for l in range(4)]
    d2p = jnp.pad(dec_W2, ((0, 0), (0, 5)))
    db2 = jnp.pad(dec_b2, (0, 5))
    f1p = jnp.pad(fc_W1, ((0, 7), (0, 0)))
    f3p = jnp.pad(fc_W3, ((0, 0), (0, 2)))
    fb3p = jnp.pad(fc_b3, (0, 2))

    p16, h, p = _tc_prep(pos_vel, enc_W8, row(enc_b), mt[0])

    gather16 = _make_sc_gather(2 * E, 16, tc_tiling=False)
    pp = gather16(p16, jnp.concatenate([src, dst]))
    ef = _tc_edge_feat(pp)

    E2 = E // 2
    gather128 = _make_sc_gather(E2, HID, ch=40)
    scatter = _make_sc_scatter(E2)
    srcs = [src[:E2], src[E2:]]
    dsts3 = [dst[:E2].reshape(NS, (E2 // NS) // CH, CH),
             dst[E2:].reshape(NS, (E2 // NS) // CH, CH)]
    zeros = jnp.zeros((ZRL, HC), F32)

    out = None
    for l in range(4):
        parts = []
        for k in range(2):
            gk = gather128(p, srcs[k])
            mk = _tc_edge_msg(gk, ef, k * (E2 // _BE), eW16, row(edge_b),
                              mbot[l], row(msg_b[l]))
            parts.append(scatter(mk, dsts3[k], zeros))
        if l < 3:
            h, p = _tc_update(h, parts[0], parts[1], ul[l], ur[l],
                              row(upd_b[l]), row(ln_g[l]), row(ln_b[l]),
                              mt[l + 1])
        else:
            out = _tc_final(h, parts[0], parts[1], ul[l], ur[l],
                            row(upd_b[l]), row(ln_g[l]), row(ln_b[l]),
                            dec_W1, row(dec_b1), d2p, row(db2), f1p,
                            row(fc_b1), fc_W2, row(fc_b2), f3p, row(fb3p),
                            pos_vel)
    return out


# speedup vs baseline: 1.7143x; 1.0754x over previous
import functools

import jax
import jax.numpy as jnp
from jax import lax
from jax.experimental import pallas as pl
from jax.experimental.pallas import tpu as pltpu
from jax.experimental.pallas import tpu_sc as plsc

N = 10000
E = 320000
HID = 128
BOX = 27.27
POS_MEAN = 13.635
POS_VAR = 61.97

NC = 2
NS = 16
NW = NC * NS
CH = 80
NB = 5
F32 = jnp.float32


def _softplus(x):
    return jnp.maximum(x, 0.0) + jnp.log1p(jnp.exp(-jnp.abs(x)))


def _make_sc_gather(n_rows, d, ch=CH, tc_tiling=True):
    per_w = n_rows // NW
    nch = per_w // ch
    assert per_w % ch == 0 and nch % NB == 0
    mesh = plsc.VectorSubcoreMesh(core_axis_name="c", subcore_axis_name="s")

    def body(table_hbm, idx_hbm, out_hbm, idx_v, *rest):
        bufs = rest[:NB]
        sems = rest[NB:]
        wid = lax.axis_index("s") * NC + lax.axis_index("c")
        base = pl.multiple_of(wid * per_w, 8)
        pltpu.sync_copy(idx_hbm.at[pl.ds(base, per_w)], idx_v)

        def step(i, carry):
            j0 = i * NB
            cps = []
            for b in range(NB):
                st = pl.multiple_of((j0 + b) * ch, 8)
                cps.append(pltpu.async_copy(
                    table_hbm.at[idx_v.at[pl.ds(st, ch)]], bufs[b], sems[b]))
            for b in range(NB):
                st = pl.multiple_of((j0 + b) * ch, 8)
                cps[b].wait()
                pltpu.sync_copy(bufs[b], out_hbm.at[pl.ds(base + st, ch)])
            return carry

        lax.fori_loop(0, nch // NB, step, 0)

    return pl.kernel(
        body,
        out_type=jax.ShapeDtypeStruct((n_rows, d), F32),
        mesh=mesh,
        scratch_types=(
            [pltpu.VMEM((per_w,), jnp.int32)]
            + [pltpu.VMEM((ch, d), F32) for _ in range(NB)]
            + [pltpu.SemaphoreType.DMA for _ in range(NB)]
        ),
        compiler_params=pltpu.CompilerParams(use_tc_tiling_on_sc=tc_tiling),
    )


def _make_sc_gather_sp(n_rows, ch=40):
    per_t = n_rows // NS
    nch = per_t // ch
    assert per_t % ch == 0 and nch % NB == 0
    mesh = plsc.VectorSubcoreMesh(core_axis_name="c", subcore_axis_name="s")

    def body(table_hbm, idx_hbm, out_hbm, idx_v, *rest):
        bufs = rest[:NB]
        ptab = rest[NB]
        sems = rest[NB + 1:]
        cid = lax.axis_index("c")
        sid = lax.axis_index("s")
        cb = pl.multiple_of(cid * HC, 8)
        zbase = pl.multiple_of(sid * ZRS, 8)
        pltpu.sync_copy(table_hbm.at[pl.ds(zbase, ZRL), pl.ds(cb, HC)],
                        ptab.at[pl.ds(zbase, ZRL)])
        base = pl.multiple_of(sid * per_t, 8)
        pltpu.sync_copy(idx_hbm.at[pl.ds(base, per_t)], idx_v)
        plsc.subcore_barrier()

        def step(i, carry):
            j0 = i * NB
            cps = []
            for b in range(NB):
                st = pl.multiple_of((j0 + b) * ch, 8)
                cps.append(pltpu.async_copy(
                    ptab.at[idx_v.at[pl.ds(st, ch)]], bufs[b], sems[b]))
            for b in range(NB):
                st = pl.multiple_of((j0 + b) * ch, 8)
                cps[b].wait()
                pltpu.sync_copy(
                    bufs[b], out_hbm.at[pl.ds(base + st, ch), pl.ds(cb, HC)])
            return carry

        lax.fori_loop(0, nch // NB, step, 0)

    return pl.kernel(
        body,
        out_type=jax.ShapeDtypeStruct((n_rows, HID), F32),
        mesh=mesh,
        scratch_types=(
            [pltpu.VMEM((per_t,), jnp.int32)]
            + [pltpu.VMEM((ch, HC), F32) for _ in range(NB)]
            + [pltpu.VMEM_SHARED((N, HC), F32)]
            + [pltpu.SemaphoreType.DMA for _ in range(NB)]
        ),
        compiler_params=pltpu.CompilerParams(use_tc_tiling_on_sc=False),
    )


HC = HID // 2
ZRS = 624
ZRL = 640


def _make_sc_scatter(n_edges):
    per_t = n_edges // NS
    nch = per_t // CH
    assert nch % NB == 0
    mesh = plsc.VectorSubcoreMesh(core_axis_name="c", subcore_axis_name="s")

    def body(m_hbm, dst_hbm, z_hbm, out_hbm, idx_v, *rest):
        bufs = rest[:NB]
        acc = rest[NB]
        sems = rest[NB + 1:]
        cid = lax.axis_index("c")
        sid = lax.axis_index("s")
        zbase = pl.multiple_of(sid * ZRS, 8)
        pltpu.sync_copy(z_hbm, acc.at[pl.ds(zbase, ZRL)])
        pltpu.sync_copy(dst_hbm.at[sid], idx_v)
        plsc.subcore_barrier()
        ebase = sid * per_t

        cb = pl.multiple_of(cid * HC, 8)

        def step(i, carry):
            j0 = i * NB
            cps = []
            for b in range(NB):
                st = pl.multiple_of(ebase + (j0 + b) * CH, 8)
                cps.append(pltpu.async_copy(
                    m_hbm.at[pl.ds(st, CH), pl.ds(cb, HC)], bufs[b], sems[b]))
            for b in range(NB):
                cps[b].wait()
                pltpu.sync_copy(bufs[b], acc.at[idx_v.at[j0 + b]], add=True)
            return carry

        lax.fori_loop(0, nch // NB, step, 0)
        plsc.subcore_barrier()
        pltpu.sync_copy(acc.at[pl.ds(zbase, ZRL)],
                        out_hbm.at[cid, pl.ds(zbase, ZRL)])

    return pl.kernel(
        body,
        out_type=jax.ShapeDtypeStruct((NC, N, HC), F32),
        mesh=mesh,
        scratch_types=(
            [pltpu.VMEM((nch, CH), jnp.int32)]
            + [pltpu.VMEM((CH, HC), F32) for _ in range(NB)]
            + [pltpu.VMEM_SHARED((N, HC), F32)]
            + [pltpu.SemaphoreType.DMA for _ in range(NB)]
        ),
        compiler_params=pltpu.CompilerParams(use_tc_tiling_on_sc=False),
    )


_BN = 1000
_BE = 4000

def _agg_specs():
    return [pl.BlockSpec((1, _BN, HC), lambda i: (0, i, 0)),
            pl.BlockSpec((1, _BN, HC), lambda i: (1, i, 0))]


def _full(spec_shape):
    return pl.BlockSpec(spec_shape, lambda i: tuple(0 for _ in spec_shape))


def _tc_prep(pos_vel, enc_W8, enc_b, mt0):
    def body(pv_ref, w_ref, b_ref, mt_ref, p16_ref, h_ref, p_ref):
        pv = pv_ref[...]
        pos = pv[:, 0:3] * jnp.sqrt(jnp.float32(POS_VAR)) + jnp.float32(POS_MEAN)
        z = jnp.zeros((pos.shape[0], 13), F32)
        p16_ref[...] = jnp.concatenate([pos, z], axis=1)
        pos8 = jnp.concatenate([pos, z[:, :5]], axis=1)
        h = _softplus(jnp.dot(pos8, w_ref[...], preferred_element_type=F32)
                      + b_ref[...])
        h_ref[...] = h
        p_ref[...] = jnp.dot(h, mt_ref[...], preferred_element_type=F32)

    grid = (N // _BN,)
    return pl.pallas_call(
        body,
        grid=grid,
        in_specs=[
            pl.BlockSpec((_BN, 6), lambda i: (i, 0)),
            _full((8, HID)),
            _full((1, HID)),
            _full((HID, HID)),
        ],
        out_specs=[
            pl.BlockSpec((_BN, 16), lambda i: (i, 0)),
            pl.BlockSpec((_BN, HID), lambda i: (i, 0)),
            pl.BlockSpec((_BN, HID), lambda i: (i, 0)),
        ],
        out_shape=[
            jax.ShapeDtypeStruct((N, 16), F32),
            jax.ShapeDtypeStruct((N, HID), F32),
            jax.ShapeDtypeStruct((N, HID), F32),
        ],
        compiler_params=pltpu.CompilerParams(
            dimension_semantics=("parallel",)),
    )(pos_vel, enc_W8, enc_b, mt0)


def _tc_edge_feat(pp):
    def body(ps_ref, pd_ref, ef_ref):
        d = ps_ref[...] - pd_ref[...]
        d = d - jnp.float32(BOX) * jnp.round(d / jnp.float32(BOX))
        ssum = jnp.sum(d * d, axis=1, keepdims=True) + jnp.float32(1e-12)
        dist = jnp.sqrt(ssum)
        col = lax.broadcasted_iota(jnp.int32, d.shape, 1)
        ef_ref[...] = jnp.where(col == 3, dist, d)

    grid = (E // _BE,)
    return pl.pallas_call(
        body,
        grid=grid,
        in_specs=[
            pl.BlockSpec((_BE, 16), lambda i: (i, 0)),
            pl.BlockSpec((_BE, 16), lambda i: (i + E // _BE, 0)),
        ],
        out_specs=pl.BlockSpec((_BE, 16), lambda i: (i, 0)),
        out_shape=jax.ShapeDtypeStruct((E, 16), F32),
        compiler_params=pltpu.CompilerParams(
            dimension_semantics=("parallel",)),
    )(pp, pp)


def _tc_edge_msg(g, ef, koff, eW16, eb, wb, mb):
    n_edges = g.shape[0]

    def body(g_ref, ef_ref, ew_ref, eb_ref, wb_ref, mb_ref, m_ref):
        e = _softplus(jnp.dot(ef_ref[...], ew_ref[...],
                              preferred_element_type=F32) + eb_ref[...])
        m_ref[...] = _softplus(
            g_ref[...] + jnp.dot(e, wb_ref[...], preferred_element_type=F32)
            + mb_ref[...])

    grid = (n_edges // _BE,)
    return pl.pallas_call(
        body,
        grid=grid,
        in_specs=[
            pl.BlockSpec((_BE, HID), lambda i: (i, 0)),
            pl.BlockSpec((_BE, 16), lambda i: (i + koff, 0)),
            _full((16, HID)),
            _full((1, HID)),
            _full((HID, HID)),
            _full((1, HID)),
        ],
        out_specs=pl.BlockSpec((_BE, HID), lambda i: (i, 0)),
        out_shape=jax.ShapeDtypeStruct((n_edges, HID), F32),
        compiler_params=pltpu.CompilerParams(
            dimension_semantics=("parallel",)),
    )(g, ef, eW16, eb, wb, mb)


def _ln_update(h, agg, ul_ref, ur_ref, ub_ref, g_ref, b_ref):
    u = _softplus(jnp.dot(h, ul_ref[...], preferred_element_type=F32)
                  + jnp.dot(agg, ur_ref[...], preferred_element_type=F32)
                  + ub_ref[...])
    hn = h + u
    mu = jnp.mean(hn, axis=1, keepdims=True)
    var = jnp.mean((hn - mu) * (hn - mu), axis=1, keepdims=True)
    return (hn - mu) / jnp.sqrt(var + jnp.float32(1e-5)) * g_ref[...] + b_ref[...]


def _agg_cat(a0l_ref, a0r_ref, a1l_ref, a1r_ref):
    return jnp.concatenate([a0l_ref[0] + a1l_ref[0],
                            a0r_ref[0] + a1r_ref[0]], axis=1)


def _tc_update(h, agg0, agg1, ul, ur, ub, lng, lnb, mt_next):
    def body(h_ref, a0l_ref, a0r_ref, a1l_ref, a1r_ref, ul_ref, ur_ref,
             ub_ref, g_ref, b_ref, mt_ref, h_out, p_out):
        agg = _agg_cat(a0l_ref, a0r_ref, a1l_ref, a1r_ref)
        hn = _ln_update(h_ref[...], agg, ul_ref, ur_ref, ub_ref,
                        g_ref, b_ref)
        h_out[...] = hn
        p_out[...] = jnp.dot(hn, mt_ref[...], preferred_element_type=F32)

    grid = (N // _BN,)
    return pl.pallas_call(
        body,
        grid=grid,
        in_specs=[
            pl.BlockSpec((_BN, HID), lambda i: (i, 0)),
            *_agg_specs(),
            *_agg_specs(),
            _full((HID, HID)),
            _full((HID, HID)),
            _full((1, HID)),
            _full((1, HID)),
            _full((1, HID)),
            _full((HID, HID)),
        ],
        out_specs=[
            pl.BlockSpec((_BN, HID), lambda i: (i, 0)),
            pl.BlockSpec((_BN, HID), lambda i: (i, 0)),
        ],
        out_shape=[
            jax.ShapeDtypeStruct((N, HID), F32),
            jax.ShapeDtypeStruct((N, HID), F32),
        ],
        compiler_params=pltpu.CompilerParams(
            dimension_semantics=("parallel",)),
    )(h, agg0, agg0, agg1, agg1, ul, ur, ub, lng, lnb, mt_next)


def _tc_final(h, agg0, agg1, ul, ur, ub, lng, lnb, d1, db1, d2p, db2, f1p,
              fb1, f2, fb2, f3p, fb3p, pos_vel):
    def body(h_ref, a0l_ref, a0r_ref, a1l_ref, a1r_ref, ul_ref, ur_ref,
             ub_ref, g_ref, b_ref, d1_ref, db1_ref, d2_ref, db2_ref, f1_ref,
             fb1_ref, f2_ref, fb2_ref, f3_ref, fb3_ref, pv_ref, out_ref):
        agg = _agg_cat(a0l_ref, a0r_ref, a1l_ref, a1r_ref)
        hn = _ln_update(h_ref[...], agg, ul_ref, ur_ref, ub_ref,
                        g_ref, b_ref)
        fmid = _softplus(jnp.dot(hn, d1_ref[...], preferred_element_type=F32)
                         + db1_ref[...])
        force = jnp.dot(fmid, d2_ref[...], preferred_element_type=F32) \
            + db2_ref[...]
        pv = pv_ref[...]
        x16 = jnp.concatenate(
            [pv, force[:, 0:3], jnp.zeros((pv.shape[0], 7), F32)], axis=1)
        x = _softplus(jnp.dot(x16, f1_ref[...], preferred_element_type=F32)
                      + fb1_ref[...])
        x = _softplus(jnp.dot(x, f2_ref[...], preferred_element_type=F32)
                      + fb2_ref[...])
        out = jnp.dot(x, f3_ref[...], preferred_element_type=F32) + fb3_ref[...]
        out_ref[...] = out[:, 0:6]

    grid = (N // _BN,)
    return pl.pallas_call(
        body,
        grid=grid,
        in_specs=[
            pl.BlockSpec((_BN, HID), lambda i: (i, 0)),
            *_agg_specs(),
            *_agg_specs(),
            _full((HID, HID)),
            _full((HID, HID)),
            _full((1, HID)),
            _full((1, HID)),
            _full((1, HID)),
            _full((HID, HID)),
            _full((1, HID)),
            _full((HID, 8)),
            _full((1, 8)),
            _full((16, HID)),
            _full((1, HID)),
            _full((HID, HID)),
            _full((1, HID)),
            _full((HID, 8)),
            _full((1, 8)),
            pl.BlockSpec((_BN, 6), lambda i: (i, 0)),
        ],
        out_specs=pl.BlockSpec((_BN, 6), lambda i: (i, 0)),
        out_shape=jax.ShapeDtypeStruct((N, 6), F32),
        compiler_params=pltpu.CompilerParams(
            dimension_semantics=("parallel",)),
    )(h, agg0, agg0, agg1, agg1, ul, ur, ub, lng, lnb, d1, db1, d2p, db2,
      f1p, fb1, f2, fb2, f3p, fb3p, pos_vel)


def kernel(pos_vel, t, edge_index, enc_W, enc_b, edge_W, edge_b, msg_W, msg_b,
           upd_W, upd_b, ln_g, ln_b, dec_W1, dec_b1, dec_W2, dec_b2, fc_W1,
           fc_b1, fc_W2, fc_b2, fc_W3, fc_b3):
    src = edge_index[0]
    dst = edge_index[1]

    enc_W8 = jnp.pad(enc_W, ((0, 5), (0, 0)))
    eW16 = jnp.pad(edge_W, ((0, 12), (0, 0)))
    row = lambda v: v.reshape(1, -1)
    mt = [msg_W[l][:HID] for l in range(4)]
    mbot = [msg_W[l][HID:] for l in range(4)]
    ul = [upd_W[l][:HID] for l in range(4)]
    ur = [upd_W[l][HID:] for l in range(4)]
    d2p = jnp.pad(dec_W2, ((0, 0), (0, 5)))
    db2 = jnp.pad(dec_b2, (0, 5))
    f1p = jnp.pad(fc_W1, ((0, 7), (0, 0)))
    f3p = jnp.pad(fc_W3, ((0, 0), (0, 2)))
    fb3p = jnp.pad(fc_b3, (0, 2))

    p16, h, p = _tc_prep(pos_vel, enc_W8, row(enc_b), mt[0])

    gather16 = _make_sc_gather(2 * E, 16, tc_tiling=False)
    pp = gather16(p16, jnp.concatenate([src, dst]))
    ef = _tc_edge_feat(pp)

    E2 = E // 2
    gather128 = _make_sc_gather_sp(E2, ch=40)
    scatter = _make_sc_scatter(E2)
    srcs = [src[:E2], src[E2:]]
    dsts3 = [dst[:E2].reshape(NS, (E2 // NS) // CH, CH),
             dst[E2:].reshape(NS, (E2 // NS) // CH, CH)]
    zeros = jnp.zeros((ZRL, HC), F32)

    out = None
    for l in range(4):
        parts = []
        for k in range(2):
            gk = gather128(p, srcs[k])
            mk = _tc_edge_msg(gk, ef, k * (E2 // _BE), eW16, row(edge_b),
                              mbot[l], row(msg_b[l]))
            parts.append(scatter(mk, dsts3[k], zeros))
        if l < 3:
            h, p = _tc_update(h, parts[0], parts[1], ul[l], ur[l],
                              row(upd_b[l]), row(ln_g[l]), row(ln_b[l]),
                              mt[l + 1])
        else:
            out = _tc_final(h, parts[0], parts[1], ul[l], ur[l],
                            row(upd_b[l]), row(ln_g[l]), row(ln_b[l]),
                            dec_W1, row(dec_b1), d2p, row(db2), f1p,
                            row(fc_b1), fc_W2, row(fc_b2), f3p, row(fb3p),
                            pos_vel)
    return out


# speedup vs baseline: 1.7404x; 1.0152x over previous
import functools

import jax
import jax.numpy as jnp
from jax import lax
from jax.experimental import pallas as pl
from jax.experimental.pallas import tpu as pltpu
from jax.experimental.pallas import tpu_sc as plsc

N = 10000
E = 320000
HID = 128
BOX = 27.27
POS_MEAN = 13.635
POS_VAR = 61.97

NC = 2
NS = 16
NW = NC * NS
CH = 80
NB = 5
F32 = jnp.float32


def _softplus(x):
    return jnp.maximum(x, 0.0) + jnp.log1p(jnp.exp(-jnp.abs(x)))


def _make_sc_gather(n_rows, d, ch=CH, tc_tiling=True):
    per_w = n_rows // NW
    nch = per_w // ch
    assert per_w % ch == 0 and nch % NB == 0
    mesh = plsc.VectorSubcoreMesh(core_axis_name="c", subcore_axis_name="s")

    def body(table_hbm, idx_hbm, out_hbm, idx_v, *rest):
        bufs = rest[:NB]
        sems = rest[NB:]
        wid = lax.axis_index("s") * NC + lax.axis_index("c")
        base = pl.multiple_of(wid * per_w, 8)
        pltpu.sync_copy(idx_hbm.at[pl.ds(base, per_w)], idx_v)

        def step(i, carry):
            j0 = i * NB
            cps = []
            for b in range(NB):
                st = pl.multiple_of((j0 + b) * ch, 8)
                cps.append(pltpu.async_copy(
                    table_hbm.at[idx_v.at[pl.ds(st, ch)]], bufs[b], sems[b]))
            for b in range(NB):
                st = pl.multiple_of((j0 + b) * ch, 8)
                cps[b].wait()
                pltpu.sync_copy(bufs[b], out_hbm.at[pl.ds(base + st, ch)])
            return carry

        lax.fori_loop(0, nch // NB, step, 0)

    return pl.kernel(
        body,
        out_type=jax.ShapeDtypeStruct((n_rows, d), F32),
        mesh=mesh,
        scratch_types=(
            [pltpu.VMEM((per_w,), jnp.int32)]
            + [pltpu.VMEM((ch, d), F32) for _ in range(NB)]
            + [pltpu.SemaphoreType.DMA for _ in range(NB)]
        ),
        compiler_params=pltpu.CompilerParams(use_tc_tiling_on_sc=tc_tiling),
    )


def _make_sc_gather_sp(n_rows, ch=40):
    per_t = n_rows // NS
    nch = per_t // ch
    assert per_t % ch == 0 and nch % NB == 0
    mesh = plsc.VectorSubcoreMesh(core_axis_name="c", subcore_axis_name="s")

    def body(table_hbm, idx_hbm, out_hbm, idx_v, *rest):
        bufs = rest[:NB]
        ptab = rest[NB]
        sems = rest[NB + 1:]
        cid = lax.axis_index("c")
        sid = lax.axis_index("s")
        cb = pl.multiple_of(cid * HC, 8)
        zbase = pl.multiple_of(sid * ZRS, 8)
        pltpu.sync_copy(table_hbm.at[pl.ds(zbase, ZRL), pl.ds(cb, HC)],
                        ptab.at[pl.ds(zbase, ZRL)])
        base = pl.multiple_of(sid * per_t, 8)
        pltpu.sync_copy(idx_hbm.at[pl.ds(base, per_t)], idx_v)
        plsc.subcore_barrier()

        def step(i, carry):
            j0 = i * NB
            cps = []
            for b in range(NB):
                st = pl.multiple_of((j0 + b) * ch, 8)
                cps.append(pltpu.async_copy(
                    ptab.at[idx_v.at[pl.ds(st, ch)]], bufs[b], sems[b]))
            for b in range(NB):
                st = pl.multiple_of((j0 + b) * ch, 8)
                cps[b].wait()
                pltpu.sync_copy(
                    bufs[b], out_hbm.at[pl.ds(base + st, ch), pl.ds(cb, HC)])
            return carry

        lax.fori_loop(0, nch // NB, step, 0)

    return pl.kernel(
        body,
        out_type=jax.ShapeDtypeStruct((n_rows, HID), F32),
        mesh=mesh,
        scratch_types=(
            [pltpu.VMEM((per_t,), jnp.int32)]
            + [pltpu.VMEM((ch, HC), F32) for _ in range(NB)]
            + [pltpu.VMEM_SHARED((N, HC), F32)]
            + [pltpu.SemaphoreType.DMA for _ in range(NB)]
        ),
        compiler_params=pltpu.CompilerParams(use_tc_tiling_on_sc=False),
    )


HC = HID // 2
ZRS = 624
ZRL = 640


def _make_sc_scatter(n_edges):
    per_t = n_edges // NS
    nch = per_t // CH
    assert nch % NB == 0
    mesh = plsc.VectorSubcoreMesh(core_axis_name="c", subcore_axis_name="s")

    def body(m_hbm, dst_hbm, z_hbm, out_hbm, idx_v, *rest):
        bufs = rest[:NB]
        acc = rest[NB]
        sems = rest[NB + 1:]
        cid = lax.axis_index("c")
        sid = lax.axis_index("s")
        zbase = pl.multiple_of(sid * ZRS, 8)
        pltpu.sync_copy(z_hbm, acc.at[pl.ds(zbase, ZRL)])
        pltpu.sync_copy(dst_hbm.at[sid], idx_v)
        plsc.subcore_barrier()
        ebase = sid * per_t

        cb = pl.multiple_of(cid * HC, 8)

        def step(i, carry):
            j0 = i * NB
            cps = []
            for b in range(NB):
                st = pl.multiple_of(ebase + (j0 + b) * CH, 8)
                cps.append(pltpu.async_copy(
                    m_hbm.at[pl.ds(st, CH), pl.ds(cb, HC)], bufs[b], sems[b]))
            for b in range(NB):
                cps[b].wait()
                pltpu.sync_copy(bufs[b], acc.at[idx_v.at[j0 + b]], add=True)
            return carry

        lax.fori_loop(0, nch // NB, step, 0)
        plsc.subcore_barrier()
        pltpu.sync_copy(acc.at[pl.ds(zbase, ZRL)],
                        out_hbm.at[cid, pl.ds(zbase, ZRL)])

    return pl.kernel(
        body,
        out_type=jax.ShapeDtypeStruct((NC, N, HC), F32),
        mesh=mesh,
        scratch_types=(
            [pltpu.VMEM((nch, CH), jnp.int32)]
            + [pltpu.VMEM((CH, HC), F32) for _ in range(NB)]
            + [pltpu.VMEM_SHARED((N, HC), F32)]
            + [pltpu.SemaphoreType.DMA for _ in range(NB)]
        ),
        compiler_params=pltpu.CompilerParams(use_tc_tiling_on_sc=False),
    )


_BN = 1000
_BE = 4000

def _agg_specs():
    return [pl.BlockSpec((1, _BN, HC), lambda i: (0, i, 0)),
            pl.BlockSpec((1, _BN, HC), lambda i: (1, i, 0))]


def _full(spec_shape):
    return pl.BlockSpec(spec_shape, lambda i: tuple(0 for _ in spec_shape))


def _tc_prep(pos_vel, enc_W8, enc_b, mt0):
    def body(pv_ref, w_ref, b_ref, mt_ref, p16_ref, h_ref, p_ref):
        pv = pv_ref[...]
        pos = pv[:, 0:3] * jnp.sqrt(jnp.float32(POS_VAR)) + jnp.float32(POS_MEAN)
        z = jnp.zeros((pos.shape[0], 13), F32)
        p16_ref[...] = jnp.concatenate([pos, z], axis=1)
        pos8 = jnp.concatenate([pos, z[:, :5]], axis=1)
        h = _softplus(jnp.dot(pos8, w_ref[...], preferred_element_type=F32)
                      + b_ref[...])
        h_ref[...] = h
        p_ref[...] = jnp.dot(h, mt_ref[...], preferred_element_type=F32)

    grid = (N // _BN,)
    return pl.pallas_call(
        body,
        grid=grid,
        in_specs=[
            pl.BlockSpec((_BN, 6), lambda i: (i, 0)),
            _full((8, HID)),
            _full((1, HID)),
            _full((HID, HID)),
        ],
        out_specs=[
            pl.BlockSpec((_BN, 16), lambda i: (i, 0)),
            pl.BlockSpec((_BN, HID), lambda i: (i, 0)),
            pl.BlockSpec((_BN, HID), lambda i: (i, 0)),
        ],
        out_shape=[
            jax.ShapeDtypeStruct((N, 16), F32),
            jax.ShapeDtypeStruct((N, HID), F32),
            jax.ShapeDtypeStruct((N, HID), F32),
        ],
        compiler_params=pltpu.CompilerParams(
            dimension_semantics=("parallel",)),
    )(pos_vel, enc_W8, enc_b, mt0)


def _tc_edge_feat(pp):
    def body(ps_ref, pd_ref, ef_ref):
        d = ps_ref[...] - pd_ref[...]
        d = d - jnp.float32(BOX) * jnp.round(d / jnp.float32(BOX))
        ssum = jnp.sum(d * d, axis=1, keepdims=True) + jnp.float32(1e-12)
        dist = jnp.sqrt(ssum)
        col = lax.broadcasted_iota(jnp.int32, d.shape, 1)
        ef_ref[...] = jnp.where(col == 3, dist, d)

    grid = (E // _BE,)
    return pl.pallas_call(
        body,
        grid=grid,
        in_specs=[
            pl.BlockSpec((_BE, 16), lambda i: (i, 0)),
            pl.BlockSpec((_BE, 16), lambda i: (i + E // _BE, 0)),
        ],
        out_specs=pl.BlockSpec((_BE, 16), lambda i: (i, 0)),
        out_shape=jax.ShapeDtypeStruct((E, 16), F32),
        compiler_params=pltpu.CompilerParams(
            dimension_semantics=("parallel",)),
    )(pp, pp)


def _tc_edge_msg(g, ef, koff, eW16, eb, wb, mb):
    n_edges = g.shape[0]

    def body(g_ref, ef_ref, ew_ref, eb_ref, wb_ref, mb_ref, m_ref):
        e = _softplus(jnp.dot(ef_ref[...], ew_ref[...],
                              preferred_element_type=F32) + eb_ref[...])
        m_ref[...] = _softplus(
            g_ref[...] + jnp.dot(e, wb_ref[...], preferred_element_type=F32)
            + mb_ref[...])

    grid = (n_edges // _BE,)
    return pl.pallas_call(
        body,
        grid=grid,
        in_specs=[
            pl.BlockSpec((_BE, HID), lambda i: (i, 0)),
            pl.BlockSpec((_BE, 16), lambda i: (i + koff, 0)),
            _full((16, HID)),
            _full((1, HID)),
            _full((HID, HID)),
            _full((1, HID)),
        ],
        out_specs=pl.BlockSpec((_BE, HID), lambda i: (i, 0)),
        out_shape=jax.ShapeDtypeStruct((n_edges, HID), F32),
        compiler_params=pltpu.CompilerParams(
            dimension_semantics=("parallel",)),
    )(g, ef, eW16, eb, wb, mb)


def _ln_update(h, agg, ul_ref, ur_ref, ub_ref, g_ref, b_ref):
    u = _softplus(jnp.dot(h, ul_ref[...], preferred_element_type=F32)
                  + jnp.dot(agg, ur_ref[...], preferred_element_type=F32)
                  + ub_ref[...])
    hn = h + u
    mu = jnp.mean(hn, axis=1, keepdims=True)
    var = jnp.mean((hn - mu) * (hn - mu), axis=1, keepdims=True)
    return (hn - mu) / jnp.sqrt(var + jnp.float32(1e-5)) * g_ref[...] + b_ref[...]


def _agg_cat(a0l_ref, a0r_ref, a1l_ref, a1r_ref):
    return jnp.concatenate([a0l_ref[0] + a1l_ref[0],
                            a0r_ref[0] + a1r_ref[0]], axis=1)


def _tc_update(h, agg0, agg1, ul, ur, ub, lng, lnb, mt_next):
    def body(h_ref, a0l_ref, a0r_ref, a1l_ref, a1r_ref, ul_ref, ur_ref,
             ub_ref, g_ref, b_ref, mt_ref, h_out, p_out):
        agg = _agg_cat(a0l_ref, a0r_ref, a1l_ref, a1r_ref)
        hn = _ln_update(h_ref[...], agg, ul_ref, ur_ref, ub_ref,
                        g_ref, b_ref)
        h_out[...] = hn
        p_out[...] = jnp.dot(hn, mt_ref[...], preferred_element_type=F32)

    grid = (N // _BN,)
    return pl.pallas_call(
        body,
        grid=grid,
        in_specs=[
            pl.BlockSpec((_BN, HID), lambda i: (i, 0)),
            *_agg_specs(),
            *_agg_specs(),
            _full((HID, HID)),
            _full((HID, HID)),
            _full((1, HID)),
            _full((1, HID)),
            _full((1, HID)),
            _full((HID, HID)),
        ],
        out_specs=[
            pl.BlockSpec((_BN, HID), lambda i: (i, 0)),
            pl.BlockSpec((_BN, HID), lambda i: (i, 0)),
        ],
        out_shape=[
            jax.ShapeDtypeStruct((N, HID), F32),
            jax.ShapeDtypeStruct((N, HID), F32),
        ],
        compiler_params=pltpu.CompilerParams(
            dimension_semantics=("parallel",)),
    )(h, agg0, agg0, agg1, agg1, ul, ur, ub, lng, lnb, mt_next)


def _tc_final(h, agg0, agg1, ul, ur, ub, lng, lnb, d1, db1, d2p, db2, f1p,
              fb1, f2, fb2, f3p, fb3p, pos_vel):
    def body(h_ref, a0l_ref, a0r_ref, a1l_ref, a1r_ref, ul_ref, ur_ref,
             ub_ref, g_ref, b_ref, d1_ref, db1_ref, d2_ref, db2_ref, f1_ref,
             fb1_ref, f2_ref, fb2_ref, f3_ref, fb3_ref, pv_ref, out_ref):
        agg = _agg_cat(a0l_ref, a0r_ref, a1l_ref, a1r_ref)
        hn = _ln_update(h_ref[...], agg, ul_ref, ur_ref, ub_ref,
                        g_ref, b_ref)
        fmid = _softplus(jnp.dot(hn, d1_ref[...], preferred_element_type=F32)
                         + db1_ref[...])
        force = jnp.dot(fmid, d2_ref[...], preferred_element_type=F32) \
            + db2_ref[...]
        pv = pv_ref[...]
        x16 = jnp.concatenate(
            [pv, force[:, 0:3], jnp.zeros((pv.shape[0], 7), F32)], axis=1)
        x = _softplus(jnp.dot(x16, f1_ref[...], preferred_element_type=F32)
                      + fb1_ref[...])
        x = _softplus(jnp.dot(x, f2_ref[...], preferred_element_type=F32)
                      + fb2_ref[...])
        out = jnp.dot(x, f3_ref[...], preferred_element_type=F32) + fb3_ref[...]
        out_ref[...] = out[:, 0:6]

    grid = (N // _BN,)
    return pl.pallas_call(
        body,
        grid=grid,
        in_specs=[
            pl.BlockSpec((_BN, HID), lambda i: (i, 0)),
            *_agg_specs(),
            *_agg_specs(),
            _full((HID, HID)),
            _full((HID, HID)),
            _full((1, HID)),
            _full((1, HID)),
            _full((1, HID)),
            _full((HID, HID)),
            _full((1, HID)),
            _full((HID, 8)),
            _full((1, 8)),
            _full((16, HID)),
            _full((1, HID)),
            _full((HID, HID)),
            _full((1, HID)),
            _full((HID, 8)),
            _full((1, 8)),
            pl.BlockSpec((_BN, 6), lambda i: (i, 0)),
        ],
        out_specs=pl.BlockSpec((_BN, 6), lambda i: (i, 0)),
        out_shape=jax.ShapeDtypeStruct((N, 6), F32),
        compiler_params=pltpu.CompilerParams(
            dimension_semantics=("parallel",)),
    )(h, agg0, agg0, agg1, agg1, ul, ur, ub, lng, lnb, d1, db1, d2p, db2,
      f1p, fb1, f2, fb2, f3p, fb3p, pos_vel)


def kernel(pos_vel, t, edge_index, enc_W, enc_b, edge_W, edge_b, msg_W, msg_b,
           upd_W, upd_b, ln_g, ln_b, dec_W1, dec_b1, dec_W2, dec_b2, fc_W1,
           fc_b1, fc_W2, fc_b2, fc_W3, fc_b3):
    src = edge_index[0]
    dst = edge_index[1]

    enc_W8 = jnp.pad(enc_W, ((0, 5), (0, 0)))
    eW16 = jnp.pad(edge_W, ((0, 12), (0, 0)))
    row = lambda v: v.reshape(1, -1)
    mt = [msg_W[l][:HID] for l in range(4)]
    mbot = [msg_W[l][HID:] for l in range(4)]
    ul = [upd_W[l][:HID] for l in range(4)]
    ur = [upd_W[l][HID:] for l in range(4)]
    d2p = jnp.pad(dec_W2, ((0, 0), (0, 5)))
    db2 = jnp.pad(dec_b2, (0, 5))
    f1p = jnp.pad(fc_W1, ((0, 7), (0, 0)))
    f3p = jnp.pad(fc_W3, ((0, 0), (0, 2)))
    fb3p = jnp.pad(fc_b3, (0, 2))

    p16, h, p = _tc_prep(pos_vel, enc_W8, row(enc_b), mt[0])

    gather16 = _make_sc_gather(2 * E, 16, tc_tiling=False)
    pp = gather16(p16, jnp.concatenate([src, dst]))
    ef = _tc_edge_feat(pp)

    E2 = E // 2
    gather128 = _make_sc_gather_sp(E2, ch=CH)
    scatter = _make_sc_scatter(E2)
    srcs = [src[:E2], src[E2:]]
    dsts3 = [dst[:E2].reshape(NS, (E2 // NS) // CH, CH),
             dst[E2:].reshape(NS, (E2 // NS) // CH, CH)]
    zeros = jnp.zeros((ZRL, HC), F32)

    out = None
    for l in range(4):
        parts = []
        for k in range(2):
            gk = gather128(p, srcs[k])
            mk = _tc_edge_msg(gk, ef, k * (E2 // _BE), eW16, row(edge_b),
                              mbot[l], row(msg_b[l]))
            parts.append(scatter(mk, dsts3[k], zeros))
        if l < 3:
            h, p = _tc_update(h, parts[0], parts[1], ul[l], ur[l],
                              row(upd_b[l]), row(ln_g[l]), row(ln_b[l]),
                              mt[l + 1])
        else:
            out = _tc_final(h, parts[0], parts[1], ul[l], ur[l],
                            row(upd_b[l]), row(ln_g[l]), row(ln_b[l]),
                            dec_W1, row(dec_b1), d2p, row(db2), f1p,
                            row(fc_b1), fc_W2, row(fc_b2), f3p, row(fb3p),
                            pos_vel)
    return out


# speedup vs baseline: 1.7404x; 1.0000x over previous
import jax
import jax.numpy as jnp
from jax import lax
from jax.experimental import pallas as pl
from jax.experimental.pallas import tpu as pltpu
from jax.experimental.pallas import tpu_sc as plsc

N = 10000
E = 320000
HID = 128
BOX = 27.27
POS_MEAN = 13.635
POS_VAR = 61.97

NC = 2
NS = 16
NW = NC * NS
CH = 80
NB = 5
F32 = jnp.float32


def _softplus(x):
    return jnp.maximum(x, 0.0) + jnp.log1p(jnp.exp(-jnp.abs(x)))


def _make_sc_gather(n_rows, d, ch=CH, tc_tiling=True):
    per_w = n_rows // NW
    nch = per_w // ch
    assert per_w % ch == 0 and nch % NB == 0
    mesh = plsc.VectorSubcoreMesh(core_axis_name="c", subcore_axis_name="s")

    def body(table_hbm, idx_hbm, out_hbm, idx_v, *rest):
        bufs = rest[:NB]
        sems = rest[NB:]
        wid = lax.axis_index("s") * NC + lax.axis_index("c")
        base = pl.multiple_of(wid * per_w, 8)
        pltpu.sync_copy(idx_hbm.at[pl.ds(base, per_w)], idx_v)

        def step(i, carry):
            j0 = i * NB
            cps = []
            for b in range(NB):
                st = pl.multiple_of((j0 + b) * ch, 8)
                cps.append(pltpu.async_copy(
                    table_hbm.at[idx_v.at[pl.ds(st, ch)]], bufs[b], sems[b]))
            for b in range(NB):
                st = pl.multiple_of((j0 + b) * ch, 8)
                cps[b].wait()
                pltpu.sync_copy(bufs[b], out_hbm.at[pl.ds(base + st, ch)])
            return carry

        lax.fori_loop(0, nch // NB, step, 0)

    return pl.kernel(
        body,
        out_type=jax.ShapeDtypeStruct((n_rows, d), F32),
        mesh=mesh,
        scratch_types=(
            [pltpu.VMEM((per_w,), jnp.int32)]
            + [pltpu.VMEM((ch, d), F32) for _ in range(NB)]
            + [pltpu.SemaphoreType.DMA for _ in range(NB)]
        ),
        compiler_params=pltpu.CompilerParams(use_tc_tiling_on_sc=tc_tiling),
    )


def _make_sc_gather_sp(n_rows, ch=40):
    per_t = n_rows // NS
    nch = per_t // ch
    assert per_t % ch == 0 and nch % NB == 0
    mesh = plsc.VectorSubcoreMesh(core_axis_name="c", subcore_axis_name="s")

    def body(table_hbm, idx_hbm, out_hbm, idx_v, *rest):
        bufs = rest[:NB]
        ptab = rest[NB]
        sems = rest[NB + 1:]
        cid = lax.axis_index("c")
        sid = lax.axis_index("s")
        cb = pl.multiple_of(cid * HC, 8)
        zbase = pl.multiple_of(sid * ZRS, 8)
        pltpu.sync_copy(table_hbm.at[pl.ds(zbase, ZRL), pl.ds(cb, HC)],
                        ptab.at[pl.ds(zbase, ZRL)])
        base = pl.multiple_of(sid * per_t, 8)
        pltpu.sync_copy(idx_hbm.at[pl.ds(base, per_t)], idx_v)
        plsc.subcore_barrier()

        def step(i, carry):
            j0 = i * NB
            cps = []
            for b in range(NB):
                st = pl.multiple_of((j0 + b) * ch, 8)
                cps.append(pltpu.async_copy(
                    ptab.at[idx_v.at[pl.ds(st, ch)]], bufs[b], sems[b]))
            for b in range(NB):
                st = pl.multiple_of((j0 + b) * ch, 8)
                cps[b].wait()
                pltpu.sync_copy(
                    bufs[b], out_hbm.at[pl.ds(base + st, ch), pl.ds(cb, HC)])
            return carry

        lax.fori_loop(0, nch // NB, step, 0)

    return pl.kernel(
        body,
        out_type=jax.ShapeDtypeStruct((n_rows, HID), F32),
        mesh=mesh,
        scratch_types=(
            [pltpu.VMEM((per_t,), jnp.int32)]
            + [pltpu.VMEM((ch, HC), F32) for _ in range(NB)]
            + [pltpu.VMEM_SHARED((N, HC), F32)]
            + [pltpu.SemaphoreType.DMA for _ in range(NB)]
        ),
        compiler_params=pltpu.CompilerParams(use_tc_tiling_on_sc=False),
    )


HC = HID // 2
ZRS = 624
ZRL = 640


def _make_sc_scatter(n_edges):
    per_t = n_edges // NS
    nch = per_t // CH
    assert nch % NB == 0
    mesh = plsc.VectorSubcoreMesh(core_axis_name="c", subcore_axis_name="s")

    def body(m_hbm, dst_hbm, z_hbm, out_hbm, idx_v, *rest):
        bufs = rest[:NB]
        acc = rest[NB]
        sems = rest[NB + 1:]
        cid = lax.axis_index("c")
        sid = lax.axis_index("s")
        zbase = pl.multiple_of(sid * ZRS, 8)
        pltpu.sync_copy(z_hbm, acc.at[pl.ds(zbase, ZRL)])
        pltpu.sync_copy(dst_hbm.at[sid], idx_v)
        plsc.subcore_barrier()
        ebase = sid * per_t

        cb = pl.multiple_of(cid * HC, 8)

        def step(i, carry):
            j0 = i * NB
            cps = []
            for b in range(NB):
                st = pl.multiple_of(ebase + (j0 + b) * CH, 8)
                cps.append(pltpu.async_copy(
                    m_hbm.at[pl.ds(st, CH), pl.ds(cb, HC)], bufs[b], sems[b]))
            for b in range(NB):
                cps[b].wait()
                pltpu.sync_copy(bufs[b], acc.at[idx_v.at[j0 + b]], add=True)
            return carry

        lax.fori_loop(0, nch // NB, step, 0)
        plsc.subcore_barrier()
        pltpu.sync_copy(acc.at[pl.ds(zbase, ZRL)],
                        out_hbm.at[cid, pl.ds(zbase, ZRL)])

    return pl.kernel(
        body,
        out_type=jax.ShapeDtypeStruct((NC, N, HC), F32),
        mesh=mesh,
        scratch_types=(
            [pltpu.VMEM((nch, CH), jnp.int32)]
            + [pltpu.VMEM((CH, HC), F32) for _ in range(NB)]
            + [pltpu.VMEM_SHARED((N, HC), F32)]
            + [pltpu.SemaphoreType.DMA for _ in range(NB)]
        ),
        compiler_params=pltpu.CompilerParams(use_tc_tiling_on_sc=False),
    )


_BN = 1000
_BE = 4000

def _agg_specs():
    return [pl.BlockSpec((1, _BN, HC), lambda i: (0, i, 0)),
            pl.BlockSpec((1, _BN, HC), lambda i: (1, i, 0))]


def _full(spec_shape):
    return pl.BlockSpec(spec_shape, lambda i: tuple(0 for _ in spec_shape))


def _tc_prep(pos_vel, enc_W8, enc_b, mt0):
    def body(pv_ref, w_ref, b_ref, mt_ref, p16_ref, h_ref, p_ref):
        pv = pv_ref[...]
        pos = pv[:, 0:3] * jnp.sqrt(jnp.float32(POS_VAR)) + jnp.float32(POS_MEAN)
        z = jnp.zeros((pos.shape[0], 13), F32)
        p16_ref[...] = jnp.concatenate([pos, z], axis=1)
        pos8 = jnp.concatenate([pos, z[:, :5]], axis=1)
        h = _softplus(jnp.dot(pos8, w_ref[...], preferred_element_type=F32)
                      + b_ref[...])
        h_ref[...] = h
        p_ref[...] = jnp.dot(h, mt_ref[...], preferred_element_type=F32)

    grid = (N // _BN,)
    return pl.pallas_call(
        body,
        grid=grid,
        in_specs=[
            pl.BlockSpec((_BN, 6), lambda i: (i, 0)),
            _full((8, HID)),
            _full((1, HID)),
            _full((HID, HID)),
        ],
        out_specs=[
            pl.BlockSpec((_BN, 16), lambda i: (i, 0)),
            pl.BlockSpec((_BN, HID), lambda i: (i, 0)),
            pl.BlockSpec((_BN, HID), lambda i: (i, 0)),
        ],
        out_shape=[
            jax.ShapeDtypeStruct((N, 16), F32),
            jax.ShapeDtypeStruct((N, HID), F32),
            jax.ShapeDtypeStruct((N, HID), F32),
        ],
        compiler_params=pltpu.CompilerParams(
            dimension_semantics=("parallel",)),
    )(pos_vel, enc_W8, enc_b, mt0)


def _tc_edge_feat(pp):
    def body(ps_ref, pd_ref, ef_ref):
        d = ps_ref[...] - pd_ref[...]
        d = d - jnp.float32(BOX) * jnp.round(d / jnp.float32(BOX))
        ssum = jnp.sum(d * d, axis=1, keepdims=True) + jnp.float32(1e-12)
        dist = jnp.sqrt(ssum)
        col = lax.broadcasted_iota(jnp.int32, d.shape, 1)
        ef_ref[...] = jnp.where(col == 3, dist, d)

    grid = (E // _BE,)
    return pl.pallas_call(
        body,
        grid=grid,
        in_specs=[
            pl.BlockSpec((_BE, 16), lambda i: (i, 0)),
            pl.BlockSpec((_BE, 16), lambda i: (i + E // _BE, 0)),
        ],
        out_specs=pl.BlockSpec((_BE, 16), lambda i: (i, 0)),
        out_shape=jax.ShapeDtypeStruct((E, 16), F32),
        compiler_params=pltpu.CompilerParams(
            dimension_semantics=("parallel",)),
    )(pp, pp)


def _tc_edge_msg(g, ef, koff, eW16, eb, wb, mb):
    n_edges = g.shape[0]

    def body(g_ref, ef_ref, ew_ref, eb_ref, wb_ref, mb_ref, m_ref):
        e = _softplus(jnp.dot(ef_ref[...], ew_ref[...],
                              preferred_element_type=F32) + eb_ref[...])
        m_ref[...] = _softplus(
            g_ref[...] + jnp.dot(e, wb_ref[...], preferred_element_type=F32)
            + mb_ref[...])

    grid = (n_edges // _BE,)
    return pl.pallas_call(
        body,
        grid=grid,
        in_specs=[
            pl.BlockSpec((_BE, HID), lambda i: (i, 0)),
            pl.BlockSpec((_BE, 16), lambda i: (i + koff, 0)),
            _full((16, HID)),
            _full((1, HID)),
            _full((HID, HID)),
            _full((1, HID)),
        ],
        out_specs=pl.BlockSpec((_BE, HID), lambda i: (i, 0)),
        out_shape=jax.ShapeDtypeStruct((n_edges, HID), F32),
        compiler_params=pltpu.CompilerParams(
            dimension_semantics=("parallel",)),
    )(g, ef, eW16, eb, wb, mb)


def _ln_update(h, agg, ul_ref, ur_ref, ub_ref, g_ref, b_ref):
    u = _softplus(jnp.dot(h, ul_ref[...], preferred_element_type=F32)
                  + jnp.dot(agg, ur_ref[...], preferred_element_type=F32)
                  + ub_ref[...])
    hn = h + u
    mu = jnp.mean(hn, axis=1, keepdims=True)
    var = jnp.mean((hn - mu) * (hn - mu), axis=1, keepdims=True)
    return (hn - mu) / jnp.sqrt(var + jnp.float32(1e-5)) * g_ref[...] + b_ref[...]


def _agg_cat(a0l_ref, a0r_ref, a1l_ref, a1r_ref):
    return jnp.concatenate([a0l_ref[0] + a1l_ref[0],
                            a0r_ref[0] + a1r_ref[0]], axis=1)


def _tc_update(h, agg0, agg1, ul, ur, ub, lng, lnb, mt_next):
    def body(h_ref, a0l_ref, a0r_ref, a1l_ref, a1r_ref, ul_ref, ur_ref,
             ub_ref, g_ref, b_ref, mt_ref, h_out, p_out):
        agg = _agg_cat(a0l_ref, a0r_ref, a1l_ref, a1r_ref)
        hn = _ln_update(h_ref[...], agg, ul_ref, ur_ref, ub_ref,
                        g_ref, b_ref)
        h_out[...] = hn
        p_out[...] = jnp.dot(hn, mt_ref[...], preferred_element_type=F32)

    grid = (N // _BN,)
    return pl.pallas_call(
        body,
        grid=grid,
        in_specs=[
            pl.BlockSpec((_BN, HID), lambda i: (i, 0)),
            *_agg_specs(),
            *_agg_specs(),
            _full((HID, HID)),
            _full((HID, HID)),
            _full((1, HID)),
            _full((1, HID)),
            _full((1, HID)),
            _full((HID, HID)),
        ],
        out_specs=[
            pl.BlockSpec((_BN, HID), lambda i: (i, 0)),
            pl.BlockSpec((_BN, HID), lambda i: (i, 0)),
        ],
        out_shape=[
            jax.ShapeDtypeStruct((N, HID), F32),
            jax.ShapeDtypeStruct((N, HID), F32),
        ],
        compiler_params=pltpu.CompilerParams(
            dimension_semantics=("parallel",)),
    )(h, agg0, agg0, agg1, agg1, ul, ur, ub, lng, lnb, mt_next)


def _tc_final(h, agg0, agg1, ul, ur, ub, lng, lnb, d1, db1, d2p, db2, f1p,
              fb1, f2, fb2, f3p, fb3p, pos_vel):
    def body(h_ref, a0l_ref, a0r_ref, a1l_ref, a1r_ref, ul_ref, ur_ref,
             ub_ref, g_ref, b_ref, d1_ref, db1_ref, d2_ref, db2_ref, f1_ref,
             fb1_ref, f2_ref, fb2_ref, f3_ref, fb3_ref, pv_ref, out_ref):
        agg = _agg_cat(a0l_ref, a0r_ref, a1l_ref, a1r_ref)
        hn = _ln_update(h_ref[...], agg, ul_ref, ur_ref, ub_ref,
                        g_ref, b_ref)
        fmid = _softplus(jnp.dot(hn, d1_ref[...], preferred_element_type=F32)
                         + db1_ref[...])
        force = jnp.dot(fmid, d2_ref[...], preferred_element_type=F32) \
            + db2_ref[...]
        pv = pv_ref[...]
        x16 = jnp.concatenate(
            [pv, force[:, 0:3], jnp.zeros((pv.shape[0], 7), F32)], axis=1)
        x = _softplus(jnp.dot(x16, f1_ref[...], preferred_element_type=F32)
                      + fb1_ref[...])
        x = _softplus(jnp.dot(x, f2_ref[...], preferred_element_type=F32)
                      + fb2_ref[...])
        out = jnp.dot(x, f3_ref[...], preferred_element_type=F32) + fb3_ref[...]
        out_ref[...] = out[:, 0:6]

    grid = (N // _BN,)
    return pl.pallas_call(
        body,
        grid=grid,
        in_specs=[
            pl.BlockSpec((_BN, HID), lambda i: (i, 0)),
            *_agg_specs(),
            *_agg_specs(),
            _full((HID, HID)),
            _full((HID, HID)),
            _full((1, HID)),
            _full((1, HID)),
            _full((1, HID)),
            _full((HID, HID)),
            _full((1, HID)),
            _full((HID, 8)),
            _full((1, 8)),
            _full((16, HID)),
            _full((1, HID)),
            _full((HID, HID)),
            _full((1, HID)),
            _full((HID, 8)),
            _full((1, 8)),
            pl.BlockSpec((_BN, 6), lambda i: (i, 0)),
        ],
        out_specs=pl.BlockSpec((_BN, 6), lambda i: (i, 0)),
        out_shape=jax.ShapeDtypeStruct((N, 6), F32),
        compiler_params=pltpu.CompilerParams(
            dimension_semantics=("parallel",)),
    )(h, agg0, agg0, agg1, agg1, ul, ur, ub, lng, lnb, d1, db1, d2p, db2,
      f1p, fb1, f2, fb2, f3p, fb3p, pos_vel)


def kernel(pos_vel, t, edge_index, enc_W, enc_b, edge_W, edge_b, msg_W, msg_b,
           upd_W, upd_b, ln_g, ln_b, dec_W1, dec_b1, dec_W2, dec_b2, fc_W1,
           fc_b1, fc_W2, fc_b2, fc_W3, fc_b3):
    src = edge_index[0]
    dst = edge_index[1]

    enc_W8 = jnp.pad(enc_W, ((0, 5), (0, 0)))
    eW16 = jnp.pad(edge_W, ((0, 12), (0, 0)))
    row = lambda v: v.reshape(1, -1)
    mt = [msg_W[l][:HID] for l in range(4)]
    mbot = [msg_W[l][HID:] for l in range(4)]
    ul = [upd_W[l][:HID] for l in range(4)]
    ur = [upd_W[l][HID:] for l in range(4)]
    d2p = jnp.pad(dec_W2, ((0, 0), (0, 5)))
    db2 = jnp.pad(dec_b2, (0, 5))
    f1p = jnp.pad(fc_W1, ((0, 7), (0, 0)))
    f3p = jnp.pad(fc_W3, ((0, 0), (0, 2)))
    fb3p = jnp.pad(fc_b3, (0, 2))

    p16, h, p = _tc_prep(pos_vel, enc_W8, row(enc_b), mt[0])

    gather16 = _make_sc_gather(2 * E, 16, tc_tiling=False)
    pp = gather16(p16, jnp.concatenate([src, dst]))
    ef = _tc_edge_feat(pp)

    E2 = E // 2
    gather128 = _make_sc_gather_sp(E2, ch=CH)
    scatter = _make_sc_scatter(E2)
    srcs = [src[:E2], src[E2:]]
    dsts3 = [dst[:E2].reshape(NS, (E2 // NS) // CH, CH),
             dst[E2:].reshape(NS, (E2 // NS) // CH, CH)]
    zeros = jnp.zeros((ZRL, HC), F32)

    out = None
    for l in range(4):
        parts = []
        for k in range(2):
            gk = gather128(p, srcs[k])
            mk = _tc_edge_msg(gk, ef, k * (E2 // _BE), eW16, row(edge_b),
                              mbot[l], row(msg_b[l]))
            parts.append(scatter(mk, dsts3[k], zeros))
        if l < 3:
            h, p = _tc_update(h, parts[0], parts[1], ul[l], ur[l],
                              row(upd_b[l]), row(ln_g[l]), row(ln_b[l]),
                              mt[l + 1])
        else:
            out = _tc_final(h, parts[0], parts[1], ul[l], ur[l],
                            row(upd_b[l]), row(ln_g[l]), row(ln_b[l]),
                            dec_W1, row(dec_b1), d2p, row(db2), f1p,
                            row(fc_b1), fc_W2, row(fc_b2), f3p, row(fb3p),
                            pos_vel)
    return out
